# baseline (speedup 1.0000x reference)
import jax
import jax.numpy as jnp
from jax import lax
from jax.experimental import pallas as pl
from jax.experimental.pallas import tpu as pltpu
from jax.experimental.pallas import tpu_sc as plsc

H = 128
NN = 10000
E = 160000
AE = 2 * E
C5 = 10000
C6 = 10000
A5 = 5 * C5
A6 = 6 * C6
AC = A5 + A6
NNP = NN + 8
NC, NS = 2, 16
NW = NC * NS
B5 = 57344
B6 = 65536
BEG = 327680
F32 = jnp.float32


def _sc_scatter_multi(tables, zinit, *, chunk):
    arrays = [zinit]
    index_of = {id(zinit): 0}
    plan = []
    for stages in tables:
        sp = []
        for (a, idx, btot, base) in stages:
            for arr in (a, idx):
                if id(arr) not in index_of:
                    index_of[id(arr)] = len(arrays)
                    arrays.append(arr)
            share = btot // NW
            assert share % chunk == 0
            sp.append((index_of[id(a)], index_of[id(idx)], share, base))
        plan.append(sp)
    n_in = len(arrays)
    n_tbl = len(tables)
    mesh = plsc.VectorSubcoreMesh(core_axis_name="c", subcore_axis_name="s")

    def body(*refs):
        ins = refs[:n_in]
        outs = refs[n_in:n_in + n_tbl]
        i0, i1, r0, r1, table, l0, l1, s0, s1 = refs[n_in + n_tbl:]
        idx_v = (i0, i1)
        rows_v = (r0, r1)
        lsem = (l0, l1)
        ssem = (s0, s1)
        c = lax.axis_index("c")
        s = lax.axis_index("s")
        wid = s * NC + c

        for k, sp in enumerate(plan):
            @pl.when(s == 0)
            def _():
                pltpu.sync_copy(ins[0].at[c], table)

            plsc.subcore_barrier()
            for (ai, ii, share, src_base) in sp:
                src_ref = ins[ai]
                idx_ref = ins[ii]
                base = wid * share
                nit = share // chunk

                def step(jj, carry, src_ref=src_ref, idx_ref=idx_ref,
                         base=base, src_base=src_base, nit=nit):
                    for b in range(2):
                        off = base + (2 * jj + b) * chunk

                        @pl.when(jj > 0)
                        def _():
                            pltpu.make_async_copy(
                                rows_v[b], table.at[idx_v[b]],
                                ssem[b]).wait()

                        pltpu.sync_copy(idx_ref.at[pl.ds(off, chunk)],
                                        idx_v[b])
                        pltpu.async_copy(
                            src_ref.at[pl.ds(src_base + off, chunk)],
                            rows_v[b], lsem[b])
                    for b in range(2):
                        pltpu.make_async_copy(
                            src_ref.at[pl.ds(src_base + base, chunk)],
                            rows_v[b], lsem[b]).wait()
                        pltpu.async_copy(rows_v[b], table.at[idx_v[b]],
                                         ssem[b], add=True)
                    return carry

                lax.fori_loop(0, nit // 2, step, 0)
                if nit % 2 == 1:
                    off = base + (nit - 1) * chunk
                    pltpu.make_async_copy(
                        rows_v[0], table.at[idx_v[0]], ssem[0]).wait()
                    pltpu.sync_copy(idx_ref.at[pl.ds(off, chunk)], idx_v[0])
                    pltpu.sync_copy(
                        src_ref.at[pl.ds(src_base + off, chunk)], rows_v[0])
                    pltpu.async_copy(rows_v[0], table.at[idx_v[0]], ssem[0],
                                     add=True)
                for b in range(2):
                    pltpu.make_async_copy(
                        rows_v[b], table.at[idx_v[b]], ssem[b]).wait()
            plsc.subcore_barrier()

            @pl.when(s == 0)
            def _():
                pltpu.sync_copy(table, outs[k].at[c])

            if k + 1 < n_tbl:
                plsc.subcore_barrier()

    f = pl.kernel(
        body,
        out_type=[jax.ShapeDtypeStruct((NC, NNP, H), F32)] * n_tbl,
        mesh=mesh,
        scratch_types=[
            pltpu.VMEM((chunk,), jnp.int32),
            pltpu.VMEM((chunk,), jnp.int32),
            pltpu.VMEM((chunk, H), F32),
            pltpu.VMEM((chunk, H), F32),
            pltpu.VMEM_SHARED((NNP, H), F32),
            pltpu.SemaphoreType.DMA,
            pltpu.SemaphoreType.DMA,
            pltpu.SemaphoreType.DMA,
            pltpu.SemaphoreType.DMA,
        ],
    )
    return f(*arrays)


def _sc_gather_multi(stages, *, chunk, width, dtype=F32):
    arrays = []
    index_of = {}
    plan = []
    for (t, idx, btot) in stages:
        for arr in (t, idx):
            if id(arr) not in index_of:
                index_of[id(arr)] = len(arrays)
                arrays.append(arr)
        share = btot // NW
        assert share % chunk == 0
        plan.append((index_of[id(t)], index_of[id(idx)], share, btot))
    n_in = len(arrays)
    n_st = len(stages)
    mesh = plsc.VectorSubcoreMesh(core_axis_name="c", subcore_axis_name="s")

    def body(*refs):
        ins = refs[:n_in]
        outs = refs[n_in:n_in + n_st]
        i0, i1, r0, r1, g0, g1, s0, s1 = refs[n_in + n_st:]
        idx_v = (i0, i1)
        rows_v = (r0, r1)
        gsem = (g0, g1)
        ssem = (s0, s1)
        c = lax.axis_index("c")
        s = lax.axis_index("s")
        wid = s * NC + c

        for k, (ti, ii, share, _) in enumerate(plan):
            table_ref = ins[ti]
            idx_ref = ins[ii]
            out_ref = outs[k]
            base = wid * share
            nit = share // chunk

            def step(jj, carry, table_ref=table_ref, idx_ref=idx_ref,
                     out_ref=out_ref, base=base):
                for b in range(2):
                    off = base + (2 * jj + b) * chunk

                    @pl.when(jj > 0)
                    def _():
                        pltpu.make_async_copy(
                            rows_v[b], out_ref.at[pl.ds(base, chunk)],
                            ssem[b]).wait()

                    pltpu.sync_copy(idx_ref.at[pl.ds(off, chunk)], idx_v[b])
                    pltpu.async_copy(table_ref.at[idx_v[b]], rows_v[b],
                                     gsem[b])
                for b in range(2):
                    off = base + (2 * jj + b) * chunk
                    pltpu.make_async_copy(table_ref.at[idx_v[b]],
                                          rows_v[b], gsem[b]).wait()
                    pltpu.async_copy(rows_v[b],
                                     out_ref.at[pl.ds(off, chunk)], ssem[b])
                return carry

            lax.fori_loop(0, nit // 2, step, 0)
            if nit % 2 == 1:
                off = base + (nit - 1) * chunk
                pltpu.make_async_copy(
                    rows_v[0], out_ref.at[pl.ds(base, chunk)], ssem[0]).wait()
                pltpu.sync_copy(idx_ref.at[pl.ds(off, chunk)], idx_v[0])
                pltpu.async_copy(table_ref.at[idx_v[0]], rows_v[0], gsem[0])
                pltpu.make_async_copy(table_ref.at[idx_v[0]], rows_v[0],
                                      gsem[0]).wait()
                pltpu.async_copy(rows_v[0], out_ref.at[pl.ds(off, chunk)],
                                 ssem[0])
            for b in range(2):
                pltpu.make_async_copy(
                    rows_v[b], out_ref.at[pl.ds(base, chunk)], ssem[b]).wait()

    f = pl.kernel(
        body,
        out_type=[jax.ShapeDtypeStruct((btot, width), dtype)
                  for (_, _, btot) in stages],
        mesh=mesh,
        scratch_types=[
            pltpu.VMEM((chunk,), jnp.int32),
            pltpu.VMEM((chunk,), jnp.int32),
            pltpu.VMEM((chunk, width), dtype),
            pltpu.VMEM((chunk, width), dtype),
            pltpu.SemaphoreType.DMA,
            pltpu.SemaphoreType.DMA,
            pltpu.SemaphoreType.DMA,
            pltpu.SemaphoreType.DMA,
        ],
    )
    return f(*arrays)


_T_NP = 72


def _tc_add2(parts):
    def body(p_ref, o_ref):
        o_ref[...] = p_ref[0] + p_ref[1]

    return pl.pallas_call(
        body,
        grid=(NNP // _T_NP,),
        in_specs=[pl.BlockSpec((2, _T_NP, H), lambda i: (0, i, 0))],
        out_specs=pl.BlockSpec((_T_NP, H), lambda i: (i, 0)),
        out_shape=jax.ShapeDtypeStruct((NNP, H), F32),
    )(parts)


def _tc_gsum_rep(x, *, g, rows, tile, out_rows):
    def body(x_ref, o_ref):
        ri = lax.broadcasted_iota(jnp.int32, (tile, tile), 0) // g
        ci = lax.broadcasted_iota(jnp.int32, (tile, tile), 1) // g
        bmat = (ri == ci).astype(F32)
        o_ref[...] = jnp.dot(bmat, x_ref[...], preferred_element_type=F32)

    return pl.pallas_call(
        body,
        grid=(rows // tile,),
        in_specs=[pl.BlockSpec((tile, H), lambda i: (i, 0))],
        out_specs=pl.BlockSpec((tile, H), lambda i: (i, 0)),
        out_shape=jax.ShapeDtypeStruct((out_rows, H), F32),
    )(x)


def _tc_te(s2a, s2b, sr, cnt5, cnt6, t5a, t5b, t6a, t6b, wb, wc):
    def body(a_ref, b_ref, e_ref, c5_ref, c6_ref, t5a_ref, t5b_ref,
             t6a_ref, t6b_ref, wb_ref, wc_ref, t1_ref, tq_ref):
        pieces = (
            a_ref[0] + a_ref[1],
            b_ref[0] + b_ref[1],
            c5_ref[...] * t5a_ref[...] + c6_ref[...] * t6a_ref[...],
            c5_ref[...] * t5b_ref[...] + c6_ref[...] * t6b_ref[...],
            e_ref[0] + e_ref[1],
        )
        t1 = jnp.zeros((_T_NP, 2 * H), F32)
        tq = jnp.zeros((_T_NP, 2 * H), F32)
        for k, piece in enumerate(pieces):
            wbk = wb_ref[k * H:(k + 1) * H, :]
            wck = wc_ref[k * H:(k + 1) * H, :]
            t1 = t1 + jnp.dot(piece, wbk + wck, preferred_element_type=F32)
            tq = tq + jnp.dot(piece, wbk, preferred_element_type=F32)
        t1_ref[...] = t1.astype(jnp.bfloat16)
        tq_ref[...] = tq.astype(jnp.bfloat16)

    part = pl.BlockSpec((2, _T_NP, H), lambda i: (0, i, 0))
    single = pl.BlockSpec((_T_NP, H), lambda i: (i, 0))
    wspec = pl.BlockSpec((5 * H, 2 * H), lambda i: (0, 0))
    return pl.pallas_call(
        body,
        grid=(NNP // _T_NP,),
        in_specs=[part, part, part, single, single, single, single,
                  single, single, wspec, wspec],
        out_specs=[pl.BlockSpec((_T_NP, 2 * H), lambda i: (i, 0))] * 2,
        out_shape=[jax.ShapeDtypeStruct((NNP, 2 * H), jnp.bfloat16)] * 2,
    )(s2a, s2b, sr, cnt5, cnt6, t5a, t5b, t6a, t6b, wb, wc)


def _tc_mul2(x, y):
    def body(x_ref, y_ref, o_ref):
        o_ref[...] = x_ref[...] * y_ref[...]

    spec = pl.BlockSpec((_T_NP, H), lambda i: (i, 0))
    return pl.pallas_call(
        body,
        grid=(NNP // _T_NP,),
        in_specs=[spec, spec],
        out_specs=spec,
        out_shape=jax.ShapeDtypeStruct((NNP, H), F32),
    )(x, y)


def _tc_edge_pass1(er, g1, g2s, wa):
    tile = 640

    def body(er_ref, g1_ref, g2_ref, wa_ref, y_ref, st_ref):
        i = pl.program_id(0)
        y = (jnp.dot(er_ref[...], wa_ref[...], preferred_element_type=F32)
             + g1_ref[...].astype(F32) + g2_ref[...].astype(F32))
        y_ref[...] = y

        @pl.when(i == 0)
        def _():
            st_ref[...] = jnp.zeros_like(st_ref)

        st_ref[0:1, :] += jnp.sum(y, axis=0, keepdims=True)
        st_ref[1:2, :] += jnp.sum(y * y, axis=0, keepdims=True)

    return pl.pallas_call(
        body,
        grid=(AE // tile,),
        in_specs=[
            pl.BlockSpec((tile, H), lambda i: (i, 0)),
            pl.BlockSpec((tile, 2 * H), lambda i: (i, 0)),
            pl.BlockSpec((tile, 2 * H), lambda i: (i, 0)),
            pl.BlockSpec((H, 2 * H), lambda i: (0, 0)),
        ],
        out_specs=[
            pl.BlockSpec((tile, 2 * H), lambda i: (i, 0)),
            pl.BlockSpec((8, 2 * H), lambda i: (0, 0)),
        ],
        out_shape=[
            jax.ShapeDtypeStruct((AE, 2 * H), F32),
            jax.ShapeDtypeStruct((8, 2 * H), F32),
        ],
    )(er, g1, g2s, wa)


def _tc_cycle_pass1(p2a, p2b, r2a, r2b, crep, vca, vcb, vba, vbb, vr, *,
                    rows, row_off):
    tile = 400

    def body(p2a_ref, p2b_ref, r2a_ref, r2b_ref, cr_ref,
             vca_ref, vcb_ref, vba_ref, vbb_ref, vr_ref, y_ref, st_ref):
        i = pl.program_id(0)
        y = (jnp.dot(p2a_ref[...], vca_ref[...], preferred_element_type=F32)
             + jnp.dot(p2b_ref[...], vcb_ref[...], preferred_element_type=F32)
             + jnp.dot(r2a_ref[...], vba_ref[...], preferred_element_type=F32)
             + jnp.dot(r2b_ref[...], vbb_ref[...], preferred_element_type=F32)
             + jnp.dot(cr_ref[...], vr_ref[...], preferred_element_type=F32))
        y_ref[...] = y

        @pl.when(i == 0)
        def _():
            st_ref[...] = jnp.zeros_like(st_ref)

        st_ref[0:1, :] += jnp.sum(y, axis=0, keepdims=True)
        st_ref[1:2, :] += jnp.sum(y * y, axis=0, keepdims=True)

    row = pl.BlockSpec((tile, H), lambda i: (i, 0))
    woff = pl.BlockSpec((tile, H), lambda i: (i + row_off, 0))
    wmat = pl.BlockSpec((H, 2 * H), lambda i: (0, 0))
    return pl.pallas_call(
        body,
        grid=(rows // tile,),
        in_specs=[row, row, row, row, woff, wmat, wmat, wmat, wmat, wmat],
        out_specs=[
            pl.BlockSpec((tile, 2 * H), lambda i: (i, 0)),
            pl.BlockSpec((8, 2 * H), lambda i: (0, 0)),
        ],
        out_shape=[
            jax.ShapeDtypeStruct((rows, 2 * H), F32),
            jax.ShapeDtypeStruct((8, 2 * H), F32),
        ],
    )(p2a, p2b, r2a, r2b, crep, vca, vcb, vba, vbb, vr)


def _tc_pass2(y, st, w2, gg, bb, *, n_total, tile, rows):
    inv_n = 1.0 / float(n_total)

    def body(y_ref, st_ref, w2_ref, g_ref, b_ref, z_ref, st2_ref):
        i = pl.program_id(0)
        m = st_ref[0:1, :] * inv_n
        v = st_ref[1:2, :] * inv_n - m * m
        r = lax.rsqrt(v + 1e-5)
        h = jnp.maximum((y_ref[...] - m) * r * g_ref[...] + b_ref[...], 0.0)
        z = jnp.dot(h, w2_ref[...], preferred_element_type=F32)
        z_ref[...] = z

        @pl.when(i == 0)
        def _():
            st2_ref[...] = jnp.zeros_like(st2_ref)

        st2_ref[0:1, :] += jnp.sum(z, axis=0, keepdims=True)
        st2_ref[1:2, :] += jnp.sum(z * z, axis=0, keepdims=True)

    return pl.pallas_call(
        body,
        grid=(rows // tile,),
        in_specs=[
            pl.BlockSpec((tile, 2 * H), lambda i: (i, 0)),
            pl.BlockSpec((8, 2 * H), lambda i: (0, 0)),
            pl.BlockSpec((2 * H, H), lambda i: (0, 0)),
            pl.BlockSpec((1, 2 * H), lambda i: (0, 0)),
            pl.BlockSpec((1, 2 * H), lambda i: (0, 0)),
        ],
        out_specs=[
            pl.BlockSpec((tile, H), lambda i: (i, 0)),
            pl.BlockSpec((8, H), lambda i: (0, 0)),
        ],
        out_shape=[
            jax.ShapeDtypeStruct((rows, H), F32),
            jax.ShapeDtypeStruct((8, H), F32),
        ],
    )(y, st, w2, gg, bb)


def _tc_pass3(z, st2, gg, bb, *, n_total, tile, rows):
    inv_n = 1.0 / float(n_total)

    def body(z_ref, st_ref, g_ref, b_ref, o_ref):
        m = st_ref[0:1, :] * inv_n
        v = st_ref[1:2, :] * inv_n - m * m
        r = lax.rsqrt(v + 1e-5)
        o_ref[...] = jnp.maximum(
            (z_ref[...] - m) * r * g_ref[...] + b_ref[...], 0.0)

    return pl.pallas_call(
        body,
        grid=(rows // tile,),
        in_specs=[
            pl.BlockSpec((tile, H), lambda i: (i, 0)),
            pl.BlockSpec((8, H), lambda i: (0, 0)),
            pl.BlockSpec((1, H), lambda i: (0, 0)),
            pl.BlockSpec((1, H), lambda i: (0, 0)),
        ],
        out_specs=pl.BlockSpec((tile, H), lambda i: (i, 0)),
        out_shape=jax.ShapeDtypeStruct((rows, H), F32),
    )(z, st2, gg, bb)


def kernel(edge_rep, cycle_rep, edge_nodes, cycle5_nodes, cycle6_nodes,
           eW1, eg1, eb1, eW2, eg2, eb2, cW1, cg1, cb1, cW2, cg2, cb2):
    en = edge_nodes.astype(jnp.int32)
    c5 = cycle5_nodes.astype(jnp.int32)
    c6 = cycle6_nodes.astype(jnp.int32)

    padv = jnp.int32(NN)
    c5p = jnp.concatenate([c5, jnp.full((B5 - A5,), padv)])
    c6p = jnp.concatenate([c6, jnp.full((B6 - A6,), padv)])
    enp = jnp.concatenate([en, jnp.full((BEG - AE,), padv)])
    ens = en.reshape(E, 2)[:, ::-1].reshape(AE)
    ensp = jnp.concatenate([ens, jnp.full((BEG - AE,), padv)])
    c6shift = jnp.concatenate([jnp.full((B6 - A6,), padv), c6])

    zinit = jnp.zeros((NC, NNP, H), F32)

    ones_rows = jnp.ones((max(B5, B6), H), F32)

    (sc1_p,) = _sc_scatter_multi([[(edge_rep, en, AE, 0)]], zinit, chunk=80)
    ns_e = _tc_add2(sc1_p)
    (cnt5_p,) = _sc_scatter_multi([[(ones_rows, c5p, B5, 0)]], zinit,
                                  chunk=128)
    (cnt6_p,) = _sc_scatter_multi([[(ones_rows, c6p, B6, 0)]], zinit,
                                  chunk=128)
    cnt5 = _tc_add2(cnt5_p)
    cnt6 = _tc_add2(cnt6_p)
    (g5,) = _sc_gather_multi([(ns_e, c5p, B5)], chunk=256, width=H)
    (g6,) = _sc_gather_multi([(ns_e, c6p, B6)], chunk=256, width=H)
    r1_5 = _tc_gsum_rep(g5, g=5, rows=A5, tile=400, out_rows=B5)
    r1_6 = _tc_gsum_rep(g6, g=6, rows=A6, tile=480, out_rows=B6)

    (t5a_p,) = _sc_scatter_multi([[(r1_5, c5p, B5, 0)]], zinit, chunk=128)
    (t6a_p,) = _sc_scatter_multi([[(r1_6, c6p, B6, 0)]], zinit, chunk=128)
    t5a = _tc_add2(t5a_p)
    t6a = _tc_add2(t6a_p)
    t5b = _tc_mul2(cnt5, ns_e)
    t6b = _tc_mul2(cnt6, ns_e)

    (p2_5a,) = _sc_gather_multi([(t5a, c5p, B5)], chunk=256, width=H)
    (p2_5b,) = _sc_gather_multi([(t5b, c5p, B5)], chunk=256, width=H)
    (p2_6a,) = _sc_gather_multi([(t6a, c6p, B6)], chunk=256, width=H)
    (p2_6b,) = _sc_gather_multi([(t6b, c6p, B6)], chunk=256, width=H)

    r2_5a = _tc_gsum_rep(p2_5a, g=5, rows=A5, tile=400, out_rows=B5)
    r2_5b = _tc_gsum_rep(p2_5b, g=5, rows=A5, tile=400, out_rows=B5)
    r2_6a = _tc_gsum_rep(p2_6a, g=6, rows=A6, tile=480, out_rows=B6)
    r2_6b = _tc_gsum_rep(p2_6b, g=6, rows=A6, tile=480, out_rows=B6)

    (s2a,) = _sc_scatter_multi(
        [[(r2_5a, c5p, B5, 0), (r2_6a, c6p, B6, 0)]], zinit, chunk=128)
    (s2b,) = _sc_scatter_multi(
        [[(r2_5b, c5p, B5, 0), (r2_6b, c6p, B6, 0)]], zinit, chunk=128)
    (sr,) = _sc_scatter_multi(
        [[(cycle_rep, c5p, B5, 0), (cycle_rep, c6shift, B6, AC - B6)]],
        zinit, chunk=128)

    wb = eW1[H:6 * H, :]
    wc = eW1[6 * H:, :]
    tg1, tq = _tc_te(s2a, s2b, sr, cnt5, cnt6, t5a, t5b, t6a, t6b, wb, wc)

    tg1i = lax.bitcast_convert_type(tg1.reshape(NNP, H, 2), jnp.int32)
    tqi = lax.bitcast_convert_type(tq.reshape(NNP, H, 2), jnp.int32)
    (g1i,) = _sc_gather_multi([(tg1i, enp, BEG)], chunk=320, width=H,
                              dtype=jnp.int32)
    (g2si,) = _sc_gather_multi([(tqi, ensp, BEG)], chunk=320, width=H,
                               dtype=jnp.int32)
    g1 = lax.bitcast_convert_type(g1i, jnp.bfloat16).reshape(BEG, 2 * H)
    g2s = lax.bitcast_convert_type(g2si, jnp.bfloat16).reshape(BEG, 2 * H)
    y_e, st_e = _tc_edge_pass1(edge_rep, g1, g2s, eW1[:H, :])
    z_e, st2_e = _tc_pass2(y_e, st_e, eW2, eg1.reshape(1, -1),
                           eb1.reshape(1, -1), n_total=AE, tile=640, rows=AE)
    edge_out = _tc_pass3(z_e, st2_e, eg2.reshape(1, -1), eb2.reshape(1, -1),
                         n_total=AE, tile=640, rows=AE)

    vb_a = cW1[0:H, :]
    vb_b = cW1[H:2 * H, :]
    vc_a = cW1[2 * H:3 * H, :]
    vc_b = cW1[3 * H:4 * H, :]
    vr = cW1[4 * H:, :]
    y5, st5 = _tc_cycle_pass1(p2_5a, p2_5b, r2_5a, r2_5b, cycle_rep,
                              vc_a, vc_b, vb_a, vb_b, vr,
                              rows=A5, row_off=0)
    y6, st6 = _tc_cycle_pass1(p2_6a, p2_6b, r2_6a, r2_6b, cycle_rep,
                              vc_a, vc_b, vb_a, vb_b, vr,
                              rows=A6, row_off=A5 // 400)
    st_c = st5 + st6
    cg1r = cg1.reshape(1, -1)
    cb1r = cb1.reshape(1, -1)
    z5, st2_5 = _tc_pass2(y5, st_c, cW2, cg1r, cb1r,
                          n_total=AC, tile=400, rows=A5)
    z6, st2_6 = _tc_pass2(y6, st_c, cW2, cg1r, cb1r,
                          n_total=AC, tile=400, rows=A6)
    st2_c = st2_5 + st2_6
    cg2r = cg2.reshape(1, -1)
    cb2r = cb2.reshape(1, -1)
    co5 = _tc_pass3(z5, st2_c, cg2r, cb2r, n_total=AC, tile=400, rows=A5)
    co6 = _tc_pass3(z6, st2_c, cg2r, cb2r, n_total=AC, tile=400, rows=A6)
    cycle_out = jnp.concatenate([co5, co6], axis=0)

    return edge_out, cycle_out


# speedup vs baseline: 1.4631x; 1.4631x over previous
import jax
import jax.numpy as jnp
from jax import lax
from jax.experimental import pallas as pl
from jax.experimental.pallas import tpu as pltpu
from jax.experimental.pallas import tpu_sc as plsc

H = 128
NN = 10000
E = 160000
AE = 2 * E
C5 = 10000
C6 = 10000
A5 = 5 * C5
A6 = 6 * C6
AC = A5 + A6
NNP = NN + 8
NC, NS = 2, 16
NW = NC * NS
B5 = 57344
B6 = 65536
BEG = 327680
F32 = jnp.float32


def _sc_scatter_multi(tables, zinit, *, chunk):
    arrays = [zinit]
    index_of = {id(zinit): 0}
    plan = []
    for stages in tables:
        sp = []
        for (a, idx, btot, base) in stages:
            for arr in (a, idx):
                if id(arr) not in index_of:
                    index_of[id(arr)] = len(arrays)
                    arrays.append(arr)
            share = btot // NW
            assert share % chunk == 0
            sp.append((index_of[id(a)], index_of[id(idx)], share, base))
        plan.append(sp)
    n_in = len(arrays)
    n_tbl = len(tables)
    mesh = plsc.VectorSubcoreMesh(core_axis_name="c", subcore_axis_name="s")

    def body(*refs):
        ins = refs[:n_in]
        outs = refs[n_in:n_in + n_tbl]
        i0, i1, r0, r1, table, l0, l1, s0, s1 = refs[n_in + n_tbl:]
        idx_v = (i0, i1)
        rows_v = (r0, r1)
        lsem = (l0, l1)
        ssem = (s0, s1)
        c = lax.axis_index("c")
        s = lax.axis_index("s")
        wid = s * NC + c

        for k, sp in enumerate(plan):
            @pl.when(s == 0)
            def _():
                pltpu.sync_copy(ins[0].at[c], table)

            plsc.subcore_barrier()
            for (ai, ii, share, src_base) in sp:
                src_ref = ins[ai]
                idx_ref = ins[ii]
                base = wid * share
                nit = share // chunk

                def step(jj, carry, src_ref=src_ref, idx_ref=idx_ref,
                         base=base, src_base=src_base, nit=nit):
                    for b in range(2):
                        off = base + (2 * jj + b) * chunk

                        @pl.when(jj > 0)
                        def _():
                            pltpu.make_async_copy(
                                rows_v[b], table.at[idx_v[b]],
                                ssem[b]).wait()

                        pltpu.sync_copy(idx_ref.at[pl.ds(off, chunk)],
                                        idx_v[b])
                        pltpu.async_copy(
                            src_ref.at[pl.ds(src_base + off, chunk)],
                            rows_v[b], lsem[b])
                    for b in range(2):
                        pltpu.make_async_copy(
                            src_ref.at[pl.ds(src_base + base, chunk)],
                            rows_v[b], lsem[b]).wait()
                        pltpu.async_copy(rows_v[b], table.at[idx_v[b]],
                                         ssem[b], add=True)
                    return carry

                lax.fori_loop(0, nit // 2, step, 0)
                if nit % 2 == 1:
                    off = base + (nit - 1) * chunk
                    pltpu.make_async_copy(
                        rows_v[0], table.at[idx_v[0]], ssem[0]).wait()
                    pltpu.sync_copy(idx_ref.at[pl.ds(off, chunk)], idx_v[0])
                    pltpu.sync_copy(
                        src_ref.at[pl.ds(src_base + off, chunk)], rows_v[0])
                    pltpu.async_copy(rows_v[0], table.at[idx_v[0]], ssem[0],
                                     add=True)
                for b in range(2):
                    pltpu.make_async_copy(
                        rows_v[b], table.at[idx_v[b]], ssem[b]).wait()
            plsc.subcore_barrier()

            @pl.when(s == 0)
            def _():
                pltpu.sync_copy(table, outs[k].at[c])

            if k + 1 < n_tbl:
                plsc.subcore_barrier()

    f = pl.kernel(
        body,
        out_type=[jax.ShapeDtypeStruct((NC, NNP, H), F32)] * n_tbl,
        mesh=mesh,
        scratch_types=[
            pltpu.VMEM((chunk,), jnp.int32),
            pltpu.VMEM((chunk,), jnp.int32),
            pltpu.VMEM((chunk, H), F32),
            pltpu.VMEM((chunk, H), F32),
            pltpu.VMEM_SHARED((NNP, H), F32),
            pltpu.SemaphoreType.DMA,
            pltpu.SemaphoreType.DMA,
            pltpu.SemaphoreType.DMA,
            pltpu.SemaphoreType.DMA,
        ],
    )
    return f(*arrays)


def _sc_gather_multi(stages, *, chunk, width, dtype=F32):
    arrays = []
    index_of = {}
    plan = []
    for (t, idx, btot) in stages:
        for arr in (t, idx):
            if id(arr) not in index_of:
                index_of[id(arr)] = len(arrays)
                arrays.append(arr)
        share = btot // NW
        assert share % chunk == 0
        plan.append((index_of[id(t)], index_of[id(idx)], share, btot))
    n_in = len(arrays)
    n_st = len(stages)
    mesh = plsc.VectorSubcoreMesh(core_axis_name="c", subcore_axis_name="s")

    def body(*refs):
        ins = refs[:n_in]
        outs = refs[n_in:n_in + n_st]
        i0, i1, r0, r1, g0, g1, s0, s1 = refs[n_in + n_st:]
        idx_v = (i0, i1)
        rows_v = (r0, r1)
        gsem = (g0, g1)
        ssem = (s0, s1)
        c = lax.axis_index("c")
        s = lax.axis_index("s")
        wid = s * NC + c

        for k, (ti, ii, share, _) in enumerate(plan):
            table_ref = ins[ti]
            idx_ref = ins[ii]
            out_ref = outs[k]
            base = wid * share
            nit = share // chunk

            def step(jj, carry, table_ref=table_ref, idx_ref=idx_ref,
                     out_ref=out_ref, base=base):
                for b in range(2):
                    off = base + (2 * jj + b) * chunk

                    @pl.when(jj > 0)
                    def _():
                        pltpu.make_async_copy(
                            rows_v[b], out_ref.at[pl.ds(base, chunk)],
                            ssem[b]).wait()

                    pltpu.sync_copy(idx_ref.at[pl.ds(off, chunk)], idx_v[b])
                    pltpu.async_copy(table_ref.at[idx_v[b]], rows_v[b],
                                     gsem[b])
                for b in range(2):
                    off = base + (2 * jj + b) * chunk
                    pltpu.make_async_copy(table_ref.at[idx_v[b]],
                                          rows_v[b], gsem[b]).wait()
                    pltpu.async_copy(rows_v[b],
                                     out_ref.at[pl.ds(off, chunk)], ssem[b])
                return carry

            lax.fori_loop(0, nit // 2, step, 0)
            if nit % 2 == 1:
                off = base + (nit - 1) * chunk
                pltpu.make_async_copy(
                    rows_v[0], out_ref.at[pl.ds(base, chunk)], ssem[0]).wait()
                pltpu.sync_copy(idx_ref.at[pl.ds(off, chunk)], idx_v[0])
                pltpu.async_copy(table_ref.at[idx_v[0]], rows_v[0], gsem[0])
                pltpu.make_async_copy(table_ref.at[idx_v[0]], rows_v[0],
                                      gsem[0]).wait()
                pltpu.async_copy(rows_v[0], out_ref.at[pl.ds(off, chunk)],
                                 ssem[0])
            for b in range(2):
                pltpu.make_async_copy(
                    rows_v[b], out_ref.at[pl.ds(base, chunk)], ssem[b]).wait()

    f = pl.kernel(
        body,
        out_type=[jax.ShapeDtypeStruct((btot, width), dtype)
                  for (_, _, btot) in stages],
        mesh=mesh,
        scratch_types=[
            pltpu.VMEM((chunk,), jnp.int32),
            pltpu.VMEM((chunk,), jnp.int32),
            pltpu.VMEM((chunk, width), dtype),
            pltpu.VMEM((chunk, width), dtype),
            pltpu.SemaphoreType.DMA,
            pltpu.SemaphoreType.DMA,
            pltpu.SemaphoreType.DMA,
            pltpu.SemaphoreType.DMA,
        ],
    )
    return f(*arrays)


_T_NP = 72


def _tc_add2(parts):
    def body(p_ref, o_ref):
        o_ref[...] = p_ref[0] + p_ref[1]

    return pl.pallas_call(
        body,
        grid=(NNP // _T_NP,),
        in_specs=[pl.BlockSpec((2, _T_NP, H), lambda i: (0, i, 0))],
        out_specs=pl.BlockSpec((_T_NP, H), lambda i: (i, 0)),
        out_shape=jax.ShapeDtypeStruct((NNP, H), F32),
    )(parts)


def _tc_gsum_rep(x, *, g, rows, tile, out_rows):
    def body(x_ref, o_ref):
        ri = lax.broadcasted_iota(jnp.int32, (tile, tile), 0) // g
        ci = lax.broadcasted_iota(jnp.int32, (tile, tile), 1) // g
        bmat = (ri == ci).astype(F32)
        o_ref[...] = jnp.dot(bmat, x_ref[...], preferred_element_type=F32)

    return pl.pallas_call(
        body,
        grid=(rows // tile,),
        in_specs=[pl.BlockSpec((tile, H), lambda i: (i, 0))],
        out_specs=pl.BlockSpec((tile, H), lambda i: (i, 0)),
        out_shape=jax.ShapeDtypeStruct((out_rows, H), F32),
    )(x)


def _tc_te(s2a, s2b, sr, cnt5, cnt6, t5a, t5b, t6a, t6b, wb, wc):
    def body(a_ref, b_ref, e_ref, c5_ref, c6_ref, t5a_ref, t5b_ref,
             t6a_ref, t6b_ref, wb_ref, wc_ref, t1_ref, tq_ref):
        pieces = (
            a_ref[0] + a_ref[1],
            b_ref[0] + b_ref[1],
            c5_ref[...] * t5a_ref[...] + c6_ref[...] * t6a_ref[...],
            c5_ref[...] * t5b_ref[...] + c6_ref[...] * t6b_ref[...],
            e_ref[0] + e_ref[1],
        )
        t1 = jnp.zeros((_T_NP, 2 * H), F32)
        tq = jnp.zeros((_T_NP, 2 * H), F32)
        for k, piece in enumerate(pieces):
            wbk = wb_ref[k * H:(k + 1) * H, :]
            wck = wc_ref[k * H:(k + 1) * H, :]
            t1 = t1 + jnp.dot(piece, wbk + wck, preferred_element_type=F32)
            tq = tq + jnp.dot(piece, wbk, preferred_element_type=F32)
        t1_ref[...] = t1
        tq_ref[...] = tq

    part = pl.BlockSpec((2, _T_NP, H), lambda i: (0, i, 0))
    single = pl.BlockSpec((_T_NP, H), lambda i: (i, 0))
    wspec = pl.BlockSpec((5 * H, 2 * H), lambda i: (0, 0))
    return pl.pallas_call(
        body,
        grid=(NNP // _T_NP,),
        in_specs=[part, part, part, single, single, single, single,
                  single, single, wspec, wspec],
        out_specs=[pl.BlockSpec((_T_NP, 2 * H), lambda i: (i, 0))] * 2,
        out_shape=[jax.ShapeDtypeStruct((NNP, 2 * H), F32)] * 2,
    )(s2a, s2b, sr, cnt5, cnt6, t5a, t5b, t6a, t6b, wb, wc)


def _tc_mul2(x, y):
    def body(x_ref, y_ref, o_ref):
        o_ref[...] = x_ref[...] * y_ref[...]

    spec = pl.BlockSpec((_T_NP, H), lambda i: (i, 0))
    return pl.pallas_call(
        body,
        grid=(NNP // _T_NP,),
        in_specs=[spec, spec],
        out_specs=spec,
        out_shape=jax.ShapeDtypeStruct((NNP, H), F32),
    )(x, y)


def _tc_edge_pass1(er, g1, g2s, wa, *, rows, row_off):
    tile = 640

    def body(er_ref, g1_ref, g2_ref, wa_ref, y_ref, st_ref):
        i = pl.program_id(0)
        y = (jnp.dot(er_ref[...], wa_ref[...], preferred_element_type=F32)
             + g1_ref[...] + g2_ref[...])
        y_ref[...] = y

        @pl.when(i == 0)
        def _():
            st_ref[...] = jnp.zeros_like(st_ref)

        st_ref[0:1, :] += jnp.sum(y, axis=0, keepdims=True)
        st_ref[1:2, :] += jnp.sum(y * y, axis=0, keepdims=True)

    return pl.pallas_call(
        body,
        grid=(rows // tile,),
        in_specs=[
            pl.BlockSpec((tile, H), lambda i: (i + row_off, 0)),
            pl.BlockSpec((tile, 2 * H), lambda i: (i, 0)),
            pl.BlockSpec((tile, 2 * H), lambda i: (i, 0)),
            pl.BlockSpec((H, 2 * H), lambda i: (0, 0)),
        ],
        out_specs=[
            pl.BlockSpec((tile, 2 * H), lambda i: (i, 0)),
            pl.BlockSpec((8, 2 * H), lambda i: (0, 0)),
        ],
        out_shape=[
            jax.ShapeDtypeStruct((rows, 2 * H), F32),
            jax.ShapeDtypeStruct((8, 2 * H), F32),
        ],
    )(er, g1, g2s, wa)


def _tc_cycle_pass1(p2a, p2b, r2a, r2b, crep, vca, vcb, vba, vbb, vr, *,
                    rows, row_off):
    tile = 400

    def body(p2a_ref, p2b_ref, r2a_ref, r2b_ref, cr_ref,
             vca_ref, vcb_ref, vba_ref, vbb_ref, vr_ref, y_ref, st_ref):
        i = pl.program_id(0)
        y = (jnp.dot(p2a_ref[...], vca_ref[...], preferred_element_type=F32)
             + jnp.dot(p2b_ref[...], vcb_ref[...], preferred_element_type=F32)
             + jnp.dot(r2a_ref[...], vba_ref[...], preferred_element_type=F32)
             + jnp.dot(r2b_ref[...], vbb_ref[...], preferred_element_type=F32)
             + jnp.dot(cr_ref[...], vr_ref[...], preferred_element_type=F32))
        y_ref[...] = y

        @pl.when(i == 0)
        def _():
            st_ref[...] = jnp.zeros_like(st_ref)

        st_ref[0:1, :] += jnp.sum(y, axis=0, keepdims=True)
        st_ref[1:2, :] += jnp.sum(y * y, axis=0, keepdims=True)

    row = pl.BlockSpec((tile, H), lambda i: (i, 0))
    woff = pl.BlockSpec((tile, H), lambda i: (i + row_off, 0))
    wmat = pl.BlockSpec((H, 2 * H), lambda i: (0, 0))
    return pl.pallas_call(
        body,
        grid=(rows // tile,),
        in_specs=[row, row, row, row, woff, wmat, wmat, wmat, wmat, wmat],
        out_specs=[
            pl.BlockSpec((tile, 2 * H), lambda i: (i, 0)),
            pl.BlockSpec((8, 2 * H), lambda i: (0, 0)),
        ],
        out_shape=[
            jax.ShapeDtypeStruct((rows, 2 * H), F32),
            jax.ShapeDtypeStruct((8, 2 * H), F32),
        ],
    )(p2a, p2b, r2a, r2b, crep, vca, vcb, vba, vbb, vr)


def _tc_pass2(y, st, w2, gg, bb, *, n_total, tile, rows):
    inv_n = 1.0 / float(n_total)

    def body(y_ref, st_ref, w2_ref, g_ref, b_ref, z_ref, st2_ref):
        i = pl.program_id(0)
        m = st_ref[0:1, :] * inv_n
        v = st_ref[1:2, :] * inv_n - m * m
        r = lax.rsqrt(v + 1e-5)
        h = jnp.maximum((y_ref[...] - m) * r * g_ref[...] + b_ref[...], 0.0)
        z = jnp.dot(h, w2_ref[...], preferred_element_type=F32)
        z_ref[...] = z

        @pl.when(i == 0)
        def _():
            st2_ref[...] = jnp.zeros_like(st2_ref)

        st2_ref[0:1, :] += jnp.sum(z, axis=0, keepdims=True)
        st2_ref[1:2, :] += jnp.sum(z * z, axis=0, keepdims=True)

    return pl.pallas_call(
        body,
        grid=(rows // tile,),
        in_specs=[
            pl.BlockSpec((tile, 2 * H), lambda i: (i, 0)),
            pl.BlockSpec((8, 2 * H), lambda i: (0, 0)),
            pl.BlockSpec((2 * H, H), lambda i: (0, 0)),
            pl.BlockSpec((1, 2 * H), lambda i: (0, 0)),
            pl.BlockSpec((1, 2 * H), lambda i: (0, 0)),
        ],
        out_specs=[
            pl.BlockSpec((tile, H), lambda i: (i, 0)),
            pl.BlockSpec((8, H), lambda i: (0, 0)),
        ],
        out_shape=[
            jax.ShapeDtypeStruct((rows, H), F32),
            jax.ShapeDtypeStruct((8, H), F32),
        ],
    )(y, st, w2, gg, bb)


def _tc_pass3(z, st2, gg, bb, *, n_total, tile, rows):
    inv_n = 1.0 / float(n_total)

    def body(z_ref, st_ref, g_ref, b_ref, o_ref):
        m = st_ref[0:1, :] * inv_n
        v = st_ref[1:2, :] * inv_n - m * m
        r = lax.rsqrt(v + 1e-5)
        o_ref[...] = jnp.maximum(
            (z_ref[...] - m) * r * g_ref[...] + b_ref[...], 0.0)

    return pl.pallas_call(
        body,
        grid=(rows // tile,),
        in_specs=[
            pl.BlockSpec((tile, H), lambda i: (i, 0)),
            pl.BlockSpec((8, H), lambda i: (0, 0)),
            pl.BlockSpec((1, H), lambda i: (0, 0)),
            pl.BlockSpec((1, H), lambda i: (0, 0)),
        ],
        out_specs=pl.BlockSpec((tile, H), lambda i: (i, 0)),
        out_shape=jax.ShapeDtypeStruct((rows, H), F32),
    )(z, st2, gg, bb)


def kernel(edge_rep, cycle_rep, edge_nodes, cycle5_nodes, cycle6_nodes,
           eW1, eg1, eb1, eW2, eg2, eb2, cW1, cg1, cb1, cW2, cg2, cb2):
    en = edge_nodes.astype(jnp.int32)
    c5 = cycle5_nodes.astype(jnp.int32)
    c6 = cycle6_nodes.astype(jnp.int32)

    padv = jnp.int32(NN)
    c5p = jnp.concatenate([c5, jnp.full((B5 - A5,), padv)])
    c6p = jnp.concatenate([c6, jnp.full((B6 - A6,), padv)])
    enp = jnp.concatenate([en, jnp.full((BEG - AE,), padv)])
    ens = en.reshape(E, 2)[:, ::-1].reshape(AE)
    ensp = jnp.concatenate([ens, jnp.full((BEG - AE,), padv)])
    c6shift = jnp.concatenate([jnp.full((B6 - A6,), padv), c6])

    zinit = jnp.zeros((NC, NNP, H), F32)

    ones_rows = jnp.ones((max(B5, B6), H), F32)

    (sc1_p,) = _sc_scatter_multi([[(edge_rep, en, AE, 0)]], zinit, chunk=80)
    ns_e = _tc_add2(sc1_p)
    (cnt5_p,) = _sc_scatter_multi([[(ones_rows, c5p, B5, 0)]], zinit,
                                  chunk=128)
    (cnt6_p,) = _sc_scatter_multi([[(ones_rows, c6p, B6, 0)]], zinit,
                                  chunk=128)
    cnt5 = _tc_add2(cnt5_p)
    cnt6 = _tc_add2(cnt6_p)
    (g5,) = _sc_gather_multi([(ns_e, c5p, B5)], chunk=256, width=H)
    (g6,) = _sc_gather_multi([(ns_e, c6p, B6)], chunk=256, width=H)
    r1_5 = _tc_gsum_rep(g5, g=5, rows=A5, tile=400, out_rows=B5)
    r1_6 = _tc_gsum_rep(g6, g=6, rows=A6, tile=480, out_rows=B6)

    (t5a_p,) = _sc_scatter_multi([[(r1_5, c5p, B5, 0)]], zinit, chunk=128)
    (t6a_p,) = _sc_scatter_multi([[(r1_6, c6p, B6, 0)]], zinit, chunk=128)
    t5a = _tc_add2(t5a_p)
    t6a = _tc_add2(t6a_p)
    t5b = _tc_mul2(cnt5, ns_e)
    t6b = _tc_mul2(cnt6, ns_e)

    (p2_5a,) = _sc_gather_multi([(t5a, c5p, B5)], chunk=256, width=H)
    (p2_5b,) = _sc_gather_multi([(t5b, c5p, B5)], chunk=256, width=H)
    (p2_6a,) = _sc_gather_multi([(t6a, c6p, B6)], chunk=256, width=H)
    (p2_6b,) = _sc_gather_multi([(t6b, c6p, B6)], chunk=256, width=H)

    r2_5a = _tc_gsum_rep(p2_5a, g=5, rows=A5, tile=400, out_rows=B5)
    r2_5b = _tc_gsum_rep(p2_5b, g=5, rows=A5, tile=400, out_rows=B5)
    r2_6a = _tc_gsum_rep(p2_6a, g=6, rows=A6, tile=480, out_rows=B6)
    r2_6b = _tc_gsum_rep(p2_6b, g=6, rows=A6, tile=480, out_rows=B6)

    (s2a,) = _sc_scatter_multi(
        [[(r2_5a, c5p, B5, 0), (r2_6a, c6p, B6, 0)]], zinit, chunk=128)
    (s2b,) = _sc_scatter_multi(
        [[(r2_5b, c5p, B5, 0), (r2_6b, c6p, B6, 0)]], zinit, chunk=128)
    (sr,) = _sc_scatter_multi(
        [[(cycle_rep, c5p, B5, 0), (cycle_rep, c6shift, B6, AC - B6)]],
        zinit, chunk=128)

    wb = eW1[H:6 * H, :]
    wc = eW1[6 * H:, :]
    tg1, tq = _tc_te(s2a, s2b, sr, cnt5, cnt6, t5a, t5b, t6a, t6b, wb, wc)

    BH = BEG // 2
    RH1 = BH
    RH2 = AE - BH
    wa = eW1[:H, :]
    eg1r = eg1.reshape(1, -1)
    eb1r = eb1.reshape(1, -1)
    eg2r = eg2.reshape(1, -1)
    eb2r = eb2.reshape(1, -1)
    enp_h2 = enp[BH:]
    ensp_h2 = ensp[BH:]
    (g1_h1,) = _sc_gather_multi([(tg1, en, BH)], chunk=160, width=2 * H)
    (g2_h1,) = _sc_gather_multi([(tq, ens, BH)], chunk=160, width=2 * H)
    y1, st_a = _tc_edge_pass1(edge_rep, g1_h1, g2_h1, wa,
                              rows=RH1, row_off=0)
    (g1_h2,) = _sc_gather_multi([(tg1, enp_h2, BH)], chunk=160, width=2 * H)
    (g2_h2,) = _sc_gather_multi([(tq, ensp_h2, BH)], chunk=160, width=2 * H)
    y2, st_b = _tc_edge_pass1(edge_rep, g1_h2, g2_h2, wa,
                              rows=RH2, row_off=RH1 // 640)
    st_e = st_a + st_b
    z1, st2_a = _tc_pass2(y1, st_e, eW2, eg1r, eb1r,
                          n_total=AE, tile=640, rows=RH1)
    z2, st2_b = _tc_pass2(y2, st_e, eW2, eg1r, eb1r,
                          n_total=AE, tile=640, rows=RH2)
    st2_e = st2_a + st2_b
    eo1 = _tc_pass3(z1, st2_e, eg2r, eb2r, n_total=AE, tile=640, rows=RH1)
    eo2 = _tc_pass3(z2, st2_e, eg2r, eb2r, n_total=AE, tile=640, rows=RH2)
    edge_out = jnp.concatenate([eo1, eo2], axis=0)

    vb_a = cW1[0:H, :]
    vb_b = cW1[H:2 * H, :]
    vc_a = cW1[2 * H:3 * H, :]
    vc_b = cW1[3 * H:4 * H, :]
    vr = cW1[4 * H:, :]
    y5, st5 = _tc_cycle_pass1(p2_5a, p2_5b, r2_5a, r2_5b, cycle_rep,
                              vc_a, vc_b, vb_a, vb_b, vr,
                              rows=A5, row_off=0)
    y6, st6 = _tc_cycle_pass1(p2_6a, p2_6b, r2_6a, r2_6b, cycle_rep,
                              vc_a, vc_b, vb_a, vb_b, vr,
                              rows=A6, row_off=A5 // 400)
    st_c = st5 + st6
    cg1r = cg1.reshape(1, -1)
    cb1r = cb1.reshape(1, -1)
    z5, st2_5 = _tc_pass2(y5, st_c, cW2, cg1r, cb1r,
                          n_total=AC, tile=400, rows=A5)
    z6, st2_6 = _tc_pass2(y6, st_c, cW2, cg1r, cb1r,
                          n_total=AC, tile=400, rows=A6)
    st2_c = st2_5 + st2_6
    cg2r = cg2.reshape(1, -1)
    cb2r = cb2.reshape(1, -1)
    co5 = _tc_pass3(z5, st2_c, cg2r, cb2r, n_total=AC, tile=400, rows=A5)
    co6 = _tc_pass3(z6, st2_c, cg2r, cb2r, n_total=AC, tile=400, rows=A6)
    cycle_out = jnp.concatenate([co5, co6], axis=0)

    return edge_out, cycle_out


# speedup vs baseline: 1.7188x; 1.1747x over previous
import jax
import jax.numpy as jnp
from jax import lax
from jax.experimental import pallas as pl
from jax.experimental.pallas import tpu as pltpu
from jax.experimental.pallas import tpu_sc as plsc

H = 128
NN = 10000
E = 160000
AE = 2 * E
C5 = 10000
C6 = 10000
A5 = 5 * C5
A6 = 6 * C6
AC = A5 + A6
NNP = NN + 8
NC, NS = 2, 16
NW = NC * NS
B5 = 57344
B6 = 65536
BEG = 327680
F32 = jnp.float32


def _sc_scatter_multi(tables, zinit, *, chunk):
    arrays = [zinit]
    index_of = {id(zinit): 0}
    plan = []
    for stages in tables:
        sp = []
        for (a, idx, btot, base) in stages:
            for arr in (a, idx):
                if id(arr) not in index_of:
                    index_of[id(arr)] = len(arrays)
                    arrays.append(arr)
            share = btot // NW
            assert share % chunk == 0
            sp.append((index_of[id(a)], index_of[id(idx)], share, base))
        plan.append(sp)
    n_in = len(arrays)
    n_tbl = len(tables)
    mesh = plsc.VectorSubcoreMesh(core_axis_name="c", subcore_axis_name="s")

    def body(*refs):
        ins = refs[:n_in]
        outs = refs[n_in:n_in + n_tbl]
        i0, i1, r0, r1, table, l0, l1, s0, s1 = refs[n_in + n_tbl:]
        idx_v = (i0, i1)
        rows_v = (r0, r1)
        lsem = (l0, l1)
        ssem = (s0, s1)
        c = lax.axis_index("c")
        s = lax.axis_index("s")
        wid = s * NC + c

        for k, sp in enumerate(plan):
            @pl.when(s == 0)
            def _():
                pltpu.sync_copy(ins[0].at[c], table)

            plsc.subcore_barrier()
            for (ai, ii, share, src_base) in sp:
                src_ref = ins[ai]
                idx_ref = ins[ii]
                base = wid * share
                nit = share // chunk

                def step(jj, carry, src_ref=src_ref, idx_ref=idx_ref,
                         base=base, src_base=src_base, nit=nit):
                    for b in range(2):
                        off = base + (2 * jj + b) * chunk

                        @pl.when(jj > 0)
                        def _():
                            pltpu.make_async_copy(
                                rows_v[b], table.at[idx_v[b]],
                                ssem[b]).wait()

                        pltpu.sync_copy(idx_ref.at[pl.ds(off, chunk)],
                                        idx_v[b])
                        pltpu.async_copy(
                            src_ref.at[pl.ds(src_base + off, chunk)],
                            rows_v[b], lsem[b])
                    for b in range(2):
                        pltpu.make_async_copy(
                            src_ref.at[pl.ds(src_base + base, chunk)],
                            rows_v[b], lsem[b]).wait()
                        pltpu.async_copy(rows_v[b], table.at[idx_v[b]],
                                         ssem[b], add=True)
                    return carry

                lax.fori_loop(0, nit // 2, step, 0)
                if nit % 2 == 1:
                    off = base + (nit - 1) * chunk
                    pltpu.make_async_copy(
                        rows_v[0], table.at[idx_v[0]], ssem[0]).wait()
                    pltpu.sync_copy(idx_ref.at[pl.ds(off, chunk)], idx_v[0])
                    pltpu.sync_copy(
                        src_ref.at[pl.ds(src_base + off, chunk)], rows_v[0])
                    pltpu.async_copy(rows_v[0], table.at[idx_v[0]], ssem[0],
                                     add=True)
                for b in range(2):
                    pltpu.make_async_copy(
                        rows_v[b], table.at[idx_v[b]], ssem[b]).wait()
            plsc.subcore_barrier()

            @pl.when(s == 0)
            def _():
                pltpu.sync_copy(table, outs[k].at[c])

            if k + 1 < n_tbl:
                plsc.subcore_barrier()

    f = pl.kernel(
        body,
        out_type=[jax.ShapeDtypeStruct((NC, NNP, H), F32)] * n_tbl,
        mesh=mesh,
        scratch_types=[
            pltpu.VMEM((chunk,), jnp.int32),
            pltpu.VMEM((chunk,), jnp.int32),
            pltpu.VMEM((chunk, H), F32),
            pltpu.VMEM((chunk, H), F32),
            pltpu.VMEM_SHARED((NNP, H), F32),
            pltpu.SemaphoreType.DMA,
            pltpu.SemaphoreType.DMA,
            pltpu.SemaphoreType.DMA,
            pltpu.SemaphoreType.DMA,
        ],
    )
    return f(*arrays)


def _sc_gather_multi(stages, *, chunk, width, dtype=F32):
    arrays = []
    index_of = {}
    plan = []
    for (t, idx, btot) in stages:
        for arr in (t, idx):
            if id(arr) not in index_of:
                index_of[id(arr)] = len(arrays)
                arrays.append(arr)
        share = btot // NW
        assert share % chunk == 0
        plan.append((index_of[id(t)], index_of[id(idx)], share, btot))
    n_in = len(arrays)
    n_st = len(stages)
    mesh = plsc.VectorSubcoreMesh(core_axis_name="c", subcore_axis_name="s")

    def body(*refs):
        ins = refs[:n_in]
        outs = refs[n_in:n_in + n_st]
        i0, i1, r0, r1, g0, g1, s0, s1 = refs[n_in + n_st:]
        idx_v = (i0, i1)
        rows_v = (r0, r1)
        gsem = (g0, g1)
        ssem = (s0, s1)
        c = lax.axis_index("c")
        s = lax.axis_index("s")
        wid = s * NC + c

        for k, (ti, ii, share, _) in enumerate(plan):
            table_ref = ins[ti]
            idx_ref = ins[ii]
            out_ref = outs[k]
            base = wid * share
            nit = share // chunk

            def step(jj, carry, table_ref=table_ref, idx_ref=idx_ref,
                     out_ref=out_ref, base=base):
                for b in range(2):
                    off = base + (2 * jj + b) * chunk

                    @pl.when(jj > 0)
                    def _():
                        pltpu.make_async_copy(
                            rows_v[b], out_ref.at[pl.ds(base, chunk)],
                            ssem[b]).wait()

                    pltpu.sync_copy(idx_ref.at[pl.ds(off, chunk)], idx_v[b])
                    pltpu.async_copy(table_ref.at[idx_v[b]], rows_v[b],
                                     gsem[b])
                for b in range(2):
                    off = base + (2 * jj + b) * chunk
                    pltpu.make_async_copy(table_ref.at[idx_v[b]],
                                          rows_v[b], gsem[b]).wait()
                    pltpu.async_copy(rows_v[b],
                                     out_ref.at[pl.ds(off, chunk)], ssem[b])
                return carry

            lax.fori_loop(0, nit // 2, step, 0)
            if nit % 2 == 1:
                off = base + (nit - 1) * chunk
                pltpu.make_async_copy(
                    rows_v[0], out_ref.at[pl.ds(base, chunk)], ssem[0]).wait()
                pltpu.sync_copy(idx_ref.at[pl.ds(off, chunk)], idx_v[0])
                pltpu.async_copy(table_ref.at[idx_v[0]], rows_v[0], gsem[0])
                pltpu.make_async_copy(table_ref.at[idx_v[0]], rows_v[0],
                                      gsem[0]).wait()
                pltpu.async_copy(rows_v[0], out_ref.at[pl.ds(off, chunk)],
                                 ssem[0])
            for b in range(2):
                pltpu.make_async_copy(
                    rows_v[b], out_ref.at[pl.ds(base, chunk)], ssem[b]).wait()

    f = pl.kernel(
        body,
        out_type=[jax.ShapeDtypeStruct((btot, width), dtype)
                  for (_, _, btot) in stages],
        mesh=mesh,
        scratch_types=[
            pltpu.VMEM((chunk,), jnp.int32),
            pltpu.VMEM((chunk,), jnp.int32),
            pltpu.VMEM((chunk, width), dtype),
            pltpu.VMEM((chunk, width), dtype),
            pltpu.SemaphoreType.DMA,
            pltpu.SemaphoreType.DMA,
            pltpu.SemaphoreType.DMA,
            pltpu.SemaphoreType.DMA,
        ],
    )
    return f(*arrays)


_T_NP = 72


def _tc_add2(parts):
    def body(p_ref, o_ref):
        o_ref[...] = p_ref[0] + p_ref[1]

    return pl.pallas_call(
        body,
        grid=(NNP // _T_NP,),
        in_specs=[pl.BlockSpec((2, _T_NP, H), lambda i: (0, i, 0))],
        out_specs=pl.BlockSpec((_T_NP, H), lambda i: (i, 0)),
        out_shape=jax.ShapeDtypeStruct((NNP, H), F32),
    )(parts)


def _tc_gsum_rep(x, *, g, rows, tile, out_rows):
    def body(x_ref, o_ref):
        ri = lax.broadcasted_iota(jnp.int32, (tile, tile), 0) // g
        ci = lax.broadcasted_iota(jnp.int32, (tile, tile), 1) // g
        bmat = (ri == ci).astype(F32)
        o_ref[...] = jnp.dot(bmat, x_ref[...], preferred_element_type=F32)

    return pl.pallas_call(
        body,
        grid=(rows // tile,),
        in_specs=[pl.BlockSpec((tile, H), lambda i: (i, 0))],
        out_specs=pl.BlockSpec((tile, H), lambda i: (i, 0)),
        out_shape=jax.ShapeDtypeStruct((out_rows, H), F32),
    )(x)


def _tc_te(s2a, s2b, sr, cnt5, cnt6, t5a, t5b, t6a, t6b, wb, wc):
    def body(a_ref, b_ref, e_ref, c5_ref, c6_ref, t5a_ref, t5b_ref,
             t6a_ref, t6b_ref, wb_ref, wc_ref, t12_ref):
        pieces = (
            a_ref[0] + a_ref[1],
            b_ref[0] + b_ref[1],
            c5_ref[...] * t5a_ref[...] + c6_ref[...] * t6a_ref[...],
            c5_ref[...] * t5b_ref[...] + c6_ref[...] * t6b_ref[...],
            e_ref[0] + e_ref[1],
        )
        t1 = jnp.zeros((_T_NP, 2 * H), F32)
        tq = jnp.zeros((_T_NP, 2 * H), F32)
        for k, piece in enumerate(pieces):
            wbk = wb_ref[k * H:(k + 1) * H, :]
            wck = wc_ref[k * H:(k + 1) * H, :]
            t1 = t1 + jnp.dot(piece, wbk + wck, preferred_element_type=F32)
            tq = tq + jnp.dot(piece, wbk, preferred_element_type=F32)
        t12_ref[:, 0:2 * H] = t1
        t12_ref[:, 2 * H:4 * H] = tq

    part = pl.BlockSpec((2, _T_NP, H), lambda i: (0, i, 0))
    single = pl.BlockSpec((_T_NP, H), lambda i: (i, 0))
    wspec = pl.BlockSpec((5 * H, 2 * H), lambda i: (0, 0))
    return pl.pallas_call(
        body,
        grid=(NNP // _T_NP,),
        in_specs=[part, part, part, single, single, single, single,
                  single, single, wspec, wspec],
        out_specs=pl.BlockSpec((_T_NP, 4 * H), lambda i: (i, 0)),
        out_shape=jax.ShapeDtypeStruct((NNP, 4 * H), F32),
    )(s2a, s2b, sr, cnt5, cnt6, t5a, t5b, t6a, t6b, wb, wc)


def _tc_mul2(x, y):
    def body(x_ref, y_ref, o_ref):
        o_ref[...] = x_ref[...] * y_ref[...]

    spec = pl.BlockSpec((_T_NP, H), lambda i: (i, 0))
    return pl.pallas_call(
        body,
        grid=(NNP // _T_NP,),
        in_specs=[spec, spec],
        out_specs=spec,
        out_shape=jax.ShapeDtypeStruct((NNP, H), F32),
    )(x, y)


def _tc_edge_pass1(er, g12, wa, *, rows, row_off):
    tile = 640

    def body(er_ref, g12_ref, wa_ref, y_ref, st_ref):
        i = pl.program_id(0)
        g1 = g12_ref[:, 0:2 * H]
        g2 = g12_ref[:, 2 * H:4 * H]
        up = pltpu.roll(g2, tile - 1, 0)
        dn = pltpu.roll(g2, 1, 0)
        parity = lax.broadcasted_iota(jnp.int32, (tile, 2 * H), 0) % 2
        g2s = jnp.where(parity == 0, up, dn)
        y = (jnp.dot(er_ref[...], wa_ref[...], preferred_element_type=F32)
             + g1 + g2s)
        y_ref[...] = y

        @pl.when(i == 0)
        def _():
            st_ref[...] = jnp.zeros_like(st_ref)

        st_ref[0:1, :] += jnp.sum(y, axis=0, keepdims=True)
        st_ref[1:2, :] += jnp.sum(y * y, axis=0, keepdims=True)

    return pl.pallas_call(
        body,
        grid=(rows // tile,),
        in_specs=[
            pl.BlockSpec((tile, H), lambda i: (i + row_off, 0)),
            pl.BlockSpec((tile, 4 * H), lambda i: (i, 0)),
            pl.BlockSpec((H, 2 * H), lambda i: (0, 0)),
        ],
        out_specs=[
            pl.BlockSpec((tile, 2 * H), lambda i: (i, 0)),
            pl.BlockSpec((8, 2 * H), lambda i: (0, 0)),
        ],
        out_shape=[
            jax.ShapeDtypeStruct((rows, 2 * H), F32),
            jax.ShapeDtypeStruct((8, 2 * H), F32),
        ],
    )(er, g12, wa)


def _tc_cycle_pass1(p2a, p2b, r2a, r2b, crep, vca, vcb, vba, vbb, vr, *,
                    rows, row_off):
    tile = 400

    def body(p2a_ref, p2b_ref, r2a_ref, r2b_ref, cr_ref,
             vca_ref, vcb_ref, vba_ref, vbb_ref, vr_ref, y_ref, st_ref):
        i = pl.program_id(0)
        y = (jnp.dot(p2a_ref[...], vca_ref[...], preferred_element_type=F32)
             + jnp.dot(p2b_ref[...], vcb_ref[...], preferred_element_type=F32)
             + jnp.dot(r2a_ref[...], vba_ref[...], preferred_element_type=F32)
             + jnp.dot(r2b_ref[...], vbb_ref[...], preferred_element_type=F32)
             + jnp.dot(cr_ref[...], vr_ref[...], preferred_element_type=F32))
        y_ref[...] = y

        @pl.when(i == 0)
        def _():
            st_ref[...] = jnp.zeros_like(st_ref)

        st_ref[0:1, :] += jnp.sum(y, axis=0, keepdims=True)
        st_ref[1:2, :] += jnp.sum(y * y, axis=0, keepdims=True)

    row = pl.BlockSpec((tile, H), lambda i: (i, 0))
    woff = pl.BlockSpec((tile, H), lambda i: (i + row_off, 0))
    wmat = pl.BlockSpec((H, 2 * H), lambda i: (0, 0))
    return pl.pallas_call(
        body,
        grid=(rows // tile,),
        in_specs=[row, row, row, row, woff, wmat, wmat, wmat, wmat, wmat],
        out_specs=[
            pl.BlockSpec((tile, 2 * H), lambda i: (i, 0)),
            pl.BlockSpec((8, 2 * H), lambda i: (0, 0)),
        ],
        out_shape=[
            jax.ShapeDtypeStruct((rows, 2 * H), F32),
            jax.ShapeDtypeStruct((8, 2 * H), F32),
        ],
    )(p2a, p2b, r2a, r2b, crep, vca, vcb, vba, vbb, vr)


def _tc_pass2(y, st, w2, gg, bb, *, n_total, tile, rows):
    inv_n = 1.0 / float(n_total)

    def body(y_ref, st_ref, w2_ref, g_ref, b_ref, z_ref, st2_ref):
        i = pl.program_id(0)
        m = st_ref[0:1, :] * inv_n
        v = st_ref[1:2, :] * inv_n - m * m
        r = lax.rsqrt(v + 1e-5)
        h = jnp.maximum((y_ref[...] - m) * r * g_ref[...] + b_ref[...], 0.0)
        z = jnp.dot(h, w2_ref[...], preferred_element_type=F32)
        z_ref[...] = z

        @pl.when(i == 0)
        def _():
            st2_ref[...] = jnp.zeros_like(st2_ref)

        st2_ref[0:1, :] += jnp.sum(z, axis=0, keepdims=True)
        st2_ref[1:2, :] += jnp.sum(z * z, axis=0, keepdims=True)

    return pl.pallas_call(
        body,
        grid=(rows // tile,),
        in_specs=[
            pl.BlockSpec((tile, 2 * H), lambda i: (i, 0)),
            pl.BlockSpec((8, 2 * H), lambda i: (0, 0)),
            pl.BlockSpec((2 * H, H), lambda i: (0, 0)),
            pl.BlockSpec((1, 2 * H), lambda i: (0, 0)),
            pl.BlockSpec((1, 2 * H), lambda i: (0, 0)),
        ],
        out_specs=[
            pl.BlockSpec((tile, H), lambda i: (i, 0)),
            pl.BlockSpec((8, H), lambda i: (0, 0)),
        ],
        out_shape=[
            jax.ShapeDtypeStruct((rows, H), F32),
            jax.ShapeDtypeStruct((8, H), F32),
        ],
    )(y, st, w2, gg, bb)


def _tc_pass3(z, st2, gg, bb, *, n_total, tile, rows):
    inv_n = 1.0 / float(n_total)

    def body(z_ref, st_ref, g_ref, b_ref, o_ref):
        m = st_ref[0:1, :] * inv_n
        v = st_ref[1:2, :] * inv_n - m * m
        r = lax.rsqrt(v + 1e-5)
        o_ref[...] = jnp.maximum(
            (z_ref[...] - m) * r * g_ref[...] + b_ref[...], 0.0)

    return pl.pallas_call(
        body,
        grid=(rows // tile,),
        in_specs=[
            pl.BlockSpec((tile, H), lambda i: (i, 0)),
            pl.BlockSpec((8, H), lambda i: (0, 0)),
            pl.BlockSpec((1, H), lambda i: (0, 0)),
            pl.BlockSpec((1, H), lambda i: (0, 0)),
        ],
        out_specs=pl.BlockSpec((tile, H), lambda i: (i, 0)),
        out_shape=jax.ShapeDtypeStruct((rows, H), F32),
    )(z, st2, gg, bb)


def kernel(edge_rep, cycle_rep, edge_nodes, cycle5_nodes, cycle6_nodes,
           eW1, eg1, eb1, eW2, eg2, eb2, cW1, cg1, cb1, cW2, cg2, cb2):
    en = edge_nodes.astype(jnp.int32)
    c5 = cycle5_nodes.astype(jnp.int32)
    c6 = cycle6_nodes.astype(jnp.int32)

    padv = jnp.int32(NN)
    c5p = jnp.concatenate([c5, jnp.full((B5 - A5,), padv)])
    c6p = jnp.concatenate([c6, jnp.full((B6 - A6,), padv)])
    enp = jnp.concatenate([en, jnp.full((BEG - AE,), padv)])
    ens = en.reshape(E, 2)[:, ::-1].reshape(AE)
    ensp = jnp.concatenate([ens, jnp.full((BEG - AE,), padv)])
    c6shift = jnp.concatenate([jnp.full((B6 - A6,), padv), c6])

    zinit = jnp.zeros((NC, NNP, H), F32)

    ones_rows = jnp.ones((max(B5, B6), H), F32)

    (sc1_p,) = _sc_scatter_multi([[(edge_rep, en, AE, 0)]], zinit, chunk=80)
    ns_e = _tc_add2(sc1_p)
    (cnt5_p,) = _sc_scatter_multi([[(ones_rows, c5p, B5, 0)]], zinit,
                                  chunk=128)
    (cnt6_p,) = _sc_scatter_multi([[(ones_rows, c6p, B6, 0)]], zinit,
                                  chunk=128)
    cnt5 = _tc_add2(cnt5_p)
    cnt6 = _tc_add2(cnt6_p)
    (g5,) = _sc_gather_multi([(ns_e, c5p, B5)], chunk=256, width=H)
    (g6,) = _sc_gather_multi([(ns_e, c6p, B6)], chunk=256, width=H)
    r1_5 = _tc_gsum_rep(g5, g=5, rows=A5, tile=400, out_rows=B5)
    r1_6 = _tc_gsum_rep(g6, g=6, rows=A6, tile=480, out_rows=B6)

    (t5a_p,) = _sc_scatter_multi([[(r1_5, c5p, B5, 0)]], zinit, chunk=128)
    (t6a_p,) = _sc_scatter_multi([[(r1_6, c6p, B6, 0)]], zinit, chunk=128)
    t5a = _tc_add2(t5a_p)
    t6a = _tc_add2(t6a_p)
    t5b = _tc_mul2(cnt5, ns_e)
    t6b = _tc_mul2(cnt6, ns_e)

    (p2_5a,) = _sc_gather_multi([(t5a, c5p, B5)], chunk=256, width=H)
    (p2_5b,) = _sc_gather_multi([(t5b, c5p, B5)], chunk=256, width=H)
    (p2_6a,) = _sc_gather_multi([(t6a, c6p, B6)], chunk=256, width=H)
    (p2_6b,) = _sc_gather_multi([(t6b, c6p, B6)], chunk=256, width=H)

    r2_5a = _tc_gsum_rep(p2_5a, g=5, rows=A5, tile=400, out_rows=B5)
    r2_5b = _tc_gsum_rep(p2_5b, g=5, rows=A5, tile=400, out_rows=B5)
    r2_6a = _tc_gsum_rep(p2_6a, g=6, rows=A6, tile=480, out_rows=B6)
    r2_6b = _tc_gsum_rep(p2_6b, g=6, rows=A6, tile=480, out_rows=B6)

    (s2a,) = _sc_scatter_multi(
        [[(r2_5a, c5p, B5, 0), (r2_6a, c6p, B6, 0)]], zinit, chunk=128)
    (s2b,) = _sc_scatter_multi(
        [[(r2_5b, c5p, B5, 0), (r2_6b, c6p, B6, 0)]], zinit, chunk=128)
    (sr,) = _sc_scatter_multi(
        [[(cycle_rep, c5p, B5, 0), (cycle_rep, c6shift, B6, AC - B6)]],
        zinit, chunk=128)

    wb = eW1[H:6 * H, :]
    wc = eW1[6 * H:, :]
    t12 = _tc_te(s2a, s2b, sr, cnt5, cnt6, t5a, t5b, t6a, t6b, wb, wc)

    BH = BEG // 2
    RH1 = BH
    RH2 = AE - BH
    wa = eW1[:H, :]
    eg1r = eg1.reshape(1, -1)
    eb1r = eb1.reshape(1, -1)
    eg2r = eg2.reshape(1, -1)
    eb2r = eb2.reshape(1, -1)
    enp_h2 = enp[BH:]
    (g12_h1,) = _sc_gather_multi([(t12, en, BH)], chunk=80, width=4 * H)
    y1, st_a = _tc_edge_pass1(edge_rep, g12_h1, wa, rows=RH1, row_off=0)
    (g12_h2,) = _sc_gather_multi([(t12, enp_h2, BH)], chunk=80,
                                 width=4 * H)
    y2, st_b = _tc_edge_pass1(edge_rep, g12_h2, wa,
                              rows=RH2, row_off=RH1 // 640)
    st_e = st_a + st_b
    z1, st2_a = _tc_pass2(y1, st_e, eW2, eg1r, eb1r,
                          n_total=AE, tile=640, rows=RH1)
    z2, st2_b = _tc_pass2(y2, st_e, eW2, eg1r, eb1r,
                          n_total=AE, tile=640, rows=RH2)
    st2_e = st2_a + st2_b
    eo1 = _tc_pass3(z1, st2_e, eg2r, eb2r, n_total=AE, tile=640, rows=RH1)
    eo2 = _tc_pass3(z2, st2_e, eg2r, eb2r, n_total=AE, tile=640, rows=RH2)
    edge_out = jnp.concatenate([eo1, eo2], axis=0)

    vb_a = cW1[0:H, :]
    vb_b = cW1[H:2 * H, :]
    vc_a = cW1[2 * H:3 * H, :]
    vc_b = cW1[3 * H:4 * H, :]
    vr = cW1[4 * H:, :]
    y5, st5 = _tc_cycle_pass1(p2_5a, p2_5b, r2_5a, r2_5b, cycle_rep,
                              vc_a, vc_b, vb_a, vb_b, vr,
                              rows=A5, row_off=0)
    y6, st6 = _tc_cycle_pass1(p2_6a, p2_6b, r2_6a, r2_6b, cycle_rep,
                              vc_a, vc_b, vb_a, vb_b, vr,
                              rows=A6, row_off=A5 // 400)
    st_c = st5 + st6
    cg1r = cg1.reshape(1, -1)
    cb1r = cb1.reshape(1, -1)
    z5, st2_5 = _tc_pass2(y5, st_c, cW2, cg1r, cb1r,
                          n_total=AC, tile=400, rows=A5)
    z6, st2_6 = _tc_pass2(y6, st_c, cW2, cg1r, cb1r,
                          n_total=AC, tile=400, rows=A6)
    st2_c = st2_5 + st2_6
    cg2r = cg2.reshape(1, -1)
    cb2r = cb2.reshape(1, -1)
    co5 = _tc_pass3(z5, st2_c, cg2r, cb2r, n_total=AC, tile=400, rows=A5)
    co6 = _tc_pass3(z6, st2_c, cg2r, cb2r, n_total=AC, tile=400, rows=A6)
    cycle_out = jnp.concatenate([co5, co6], axis=0)

    return edge_out, cycle_out


# speedup vs baseline: 1.9045x; 1.1081x over previous
import jax
import jax.numpy as jnp
from jax import lax
from jax.experimental import pallas as pl
from jax.experimental.pallas import tpu as pltpu
from jax.experimental.pallas import tpu_sc as plsc

H = 128
NN = 10000
E = 160000
AE = 2 * E
C5 = 10000
C6 = 10000
A5 = 5 * C5
A6 = 6 * C6
AC = A5 + A6
NNP = NN + 8
NC, NS = 2, 16
NW = NC * NS
B5 = 57344
B6 = 65536
BEG = 327680
F32 = jnp.float32


def _sc_scatter_multi(tables, zinit, *, chunk):
    arrays = [zinit]
    index_of = {id(zinit): 0}
    plan = []
    for stages in tables:
        sp = []
        for (a, idx, btot, base) in stages:
            for arr in (a, idx):
                if id(arr) not in index_of:
                    index_of[id(arr)] = len(arrays)
                    arrays.append(arr)
            share = btot // NW
            assert share % chunk == 0
            sp.append((index_of[id(a)], index_of[id(idx)], share, base))
        plan.append(sp)
    n_in = len(arrays)
    n_tbl = len(tables)
    mesh = plsc.VectorSubcoreMesh(core_axis_name="c", subcore_axis_name="s")

    def body(*refs):
        ins = refs[:n_in]
        outs = refs[n_in:n_in + n_tbl]
        i0, i1, r0, r1, table, l0, l1, s0, s1 = refs[n_in + n_tbl:]
        idx_v = (i0, i1)
        rows_v = (r0, r1)
        lsem = (l0, l1)
        ssem = (s0, s1)
        c = lax.axis_index("c")
        s = lax.axis_index("s")
        wid = s * NC + c

        for k, sp in enumerate(plan):
            @pl.when(s == 0)
            def _():
                pltpu.sync_copy(ins[0].at[c], table)

            plsc.subcore_barrier()
            for (ai, ii, share, src_base) in sp:
                src_ref = ins[ai]
                idx_ref = ins[ii]
                base = wid * share
                nit = share // chunk

                def step(jj, carry, src_ref=src_ref, idx_ref=idx_ref,
                         base=base, src_base=src_base, nit=nit):
                    for b in range(2):
                        off = base + (2 * jj + b) * chunk

                        @pl.when(jj > 0)
                        def _():
                            pltpu.make_async_copy(
                                rows_v[b], table.at[idx_v[b]],
                                ssem[b]).wait()

                        pltpu.sync_copy(idx_ref.at[pl.ds(off, chunk)],
                                        idx_v[b])
                        pltpu.async_copy(
                            src_ref.at[pl.ds(src_base + off, chunk)],
                            rows_v[b], lsem[b])
                    for b in range(2):
                        pltpu.make_async_copy(
                            src_ref.at[pl.ds(src_base + base, chunk)],
                            rows_v[b], lsem[b]).wait()
                        pltpu.async_copy(rows_v[b], table.at[idx_v[b]],
                                         ssem[b], add=True)
                    return carry

                lax.fori_loop(0, nit // 2, step, 0)
                if nit % 2 == 1:
                    off = base + (nit - 1) * chunk
                    pltpu.make_async_copy(
                        rows_v[0], table.at[idx_v[0]], ssem[0]).wait()
                    pltpu.sync_copy(idx_ref.at[pl.ds(off, chunk)], idx_v[0])
                    pltpu.sync_copy(
                        src_ref.at[pl.ds(src_base + off, chunk)], rows_v[0])
                    pltpu.async_copy(rows_v[0], table.at[idx_v[0]], ssem[0],
                                     add=True)
                for b in range(2):
                    pltpu.make_async_copy(
                        rows_v[b], table.at[idx_v[b]], ssem[b]).wait()
            plsc.subcore_barrier()

            @pl.when(s == 0)
            def _():
                pltpu.sync_copy(table, outs[k].at[c])

            if k + 1 < n_tbl:
                plsc.subcore_barrier()

    f = pl.kernel(
        body,
        out_type=[jax.ShapeDtypeStruct((NC, NNP, H), F32)] * n_tbl,
        mesh=mesh,
        scratch_types=[
            pltpu.VMEM((chunk,), jnp.int32),
            pltpu.VMEM((chunk,), jnp.int32),
            pltpu.VMEM((chunk, H), F32),
            pltpu.VMEM((chunk, H), F32),
            pltpu.VMEM_SHARED((NNP, H), F32),
            pltpu.SemaphoreType.DMA,
            pltpu.SemaphoreType.DMA,
            pltpu.SemaphoreType.DMA,
            pltpu.SemaphoreType.DMA,
        ],
    )
    return f(*arrays)


def _sc_gather_multi(stages, *, chunk, width, dtype=F32):
    arrays = []
    index_of = {}
    plan = []
    for (t, idx, btot) in stages:
        for arr in (t, idx):
            if id(arr) not in index_of:
                index_of[id(arr)] = len(arrays)
                arrays.append(arr)
        share = btot // NW
        assert share % chunk == 0
        plan.append((index_of[id(t)], index_of[id(idx)], share, btot))
    n_in = len(arrays)
    n_st = len(stages)
    mesh = plsc.VectorSubcoreMesh(core_axis_name="c", subcore_axis_name="s")

    def body(*refs):
        ins = refs[:n_in]
        outs = refs[n_in:n_in + n_st]
        i0, i1, r0, r1, g0, g1, s0, s1 = refs[n_in + n_st:]
        idx_v = (i0, i1)
        rows_v = (r0, r1)
        gsem = (g0, g1)
        ssem = (s0, s1)
        c = lax.axis_index("c")
        s = lax.axis_index("s")
        wid = s * NC + c

        for k, (ti, ii, share, _) in enumerate(plan):
            table_ref = ins[ti]
            idx_ref = ins[ii]
            out_ref = outs[k]
            base = wid * share
            nit = share // chunk

            def step(jj, carry, table_ref=table_ref, idx_ref=idx_ref,
                     out_ref=out_ref, base=base):
                for b in range(2):
                    off = base + (2 * jj + b) * chunk

                    @pl.when(jj > 0)
                    def _():
                        pltpu.make_async_copy(
                            rows_v[b], out_ref.at[pl.ds(base, chunk)],
                            ssem[b]).wait()

                    pltpu.sync_copy(idx_ref.at[pl.ds(off, chunk)], idx_v[b])
                    pltpu.async_copy(table_ref.at[idx_v[b]], rows_v[b],
                                     gsem[b])
                for b in range(2):
                    off = base + (2 * jj + b) * chunk
                    pltpu.make_async_copy(table_ref.at[idx_v[b]],
                                          rows_v[b], gsem[b]).wait()
                    pltpu.async_copy(rows_v[b],
                                     out_ref.at[pl.ds(off, chunk)], ssem[b])
                return carry

            lax.fori_loop(0, nit // 2, step, 0)
            if nit % 2 == 1:
                off = base + (nit - 1) * chunk
                pltpu.make_async_copy(
                    rows_v[0], out_ref.at[pl.ds(base, chunk)], ssem[0]).wait()
                pltpu.sync_copy(idx_ref.at[pl.ds(off, chunk)], idx_v[0])
                pltpu.async_copy(table_ref.at[idx_v[0]], rows_v[0], gsem[0])
                pltpu.make_async_copy(table_ref.at[idx_v[0]], rows_v[0],
                                      gsem[0]).wait()
                pltpu.async_copy(rows_v[0], out_ref.at[pl.ds(off, chunk)],
                                 ssem[0])
            for b in range(2):
                pltpu.make_async_copy(
                    rows_v[b], out_ref.at[pl.ds(base, chunk)], ssem[b]).wait()

    f = pl.kernel(
        body,
        out_type=[jax.ShapeDtypeStruct((btot, width), dtype)
                  for (_, _, btot) in stages],
        mesh=mesh,
        scratch_types=[
            pltpu.VMEM((chunk,), jnp.int32),
            pltpu.VMEM((chunk,), jnp.int32),
            pltpu.VMEM((chunk, width), dtype),
            pltpu.VMEM((chunk, width), dtype),
            pltpu.SemaphoreType.DMA,
            pltpu.SemaphoreType.DMA,
            pltpu.SemaphoreType.DMA,
            pltpu.SemaphoreType.DMA,
        ],
    )
    return f(*arrays)


_T_NP = 72


def _tc_add2(parts):
    def body(p_ref, o_ref):
        o_ref[...] = p_ref[0] + p_ref[1]

    return pl.pallas_call(
        body,
        grid=(NNP // _T_NP,),
        in_specs=[pl.BlockSpec((2, _T_NP, H), lambda i: (0, i, 0))],
        out_specs=pl.BlockSpec((_T_NP, H), lambda i: (i, 0)),
        out_shape=jax.ShapeDtypeStruct((NNP, H), F32),
    )(parts)


def _tc_gsum_rep2(x, *, g, rows, tile, out_rows):
    def body(x_ref, o1_ref, o2_ref):
        ri = lax.broadcasted_iota(jnp.int32, (tile, tile), 0) // g
        ci = lax.broadcasted_iota(jnp.int32, (tile, tile), 1) // g
        bmat = (ri == ci).astype(F32)
        rep = jnp.dot(bmat, x_ref[...], preferred_element_type=F32)
        o1_ref[...] = rep[:, 0:H]
        o2_ref[...] = rep[:, H:2 * H]

    return pl.pallas_call(
        body,
        grid=(rows // tile,),
        in_specs=[pl.BlockSpec((tile, 2 * H), lambda i: (i, 0))],
        out_specs=[pl.BlockSpec((tile, H), lambda i: (i, 0))] * 2,
        out_shape=[jax.ShapeDtypeStruct((out_rows, H), F32)] * 2,
    )(x)


def _tc_gsum_rep(x, *, g, rows, tile, out_rows):
    def body(x_ref, o_ref):
        ri = lax.broadcasted_iota(jnp.int32, (tile, tile), 0) // g
        ci = lax.broadcasted_iota(jnp.int32, (tile, tile), 1) // g
        bmat = (ri == ci).astype(F32)
        o_ref[...] = jnp.dot(bmat, x_ref[...], preferred_element_type=F32)

    return pl.pallas_call(
        body,
        grid=(rows // tile,),
        in_specs=[pl.BlockSpec((tile, H), lambda i: (i, 0))],
        out_specs=pl.BlockSpec((tile, H), lambda i: (i, 0)),
        out_shape=jax.ShapeDtypeStruct((out_rows, H), F32),
    )(x)


def _tc_te(s2a, s2b, sr, cnt5, cnt6, t5c, t6c, wb, wc):
    def body(a_ref, b_ref, e_ref, c5_ref, c6_ref, t5_ref, t6_ref,
             wb_ref, wc_ref, t12_ref):
        pieces = (
            a_ref[0] + a_ref[1],
            b_ref[0] + b_ref[1],
            c5_ref[...] * t5_ref[:, 0:H] + c6_ref[...] * t6_ref[:, 0:H],
            c5_ref[...] * t5_ref[:, H:2 * H]
            + c6_ref[...] * t6_ref[:, H:2 * H],
            e_ref[0] + e_ref[1],
        )
        t1 = jnp.zeros((_T_NP, 2 * H), F32)
        tq = jnp.zeros((_T_NP, 2 * H), F32)
        for k, piece in enumerate(pieces):
            wbk = wb_ref[k * H:(k + 1) * H, :]
            wck = wc_ref[k * H:(k + 1) * H, :]
            t1 = t1 + jnp.dot(piece, wbk + wck, preferred_element_type=F32)
            tq = tq + jnp.dot(piece, wbk, preferred_element_type=F32)
        t12_ref[:, 0:2 * H] = t1
        t12_ref[:, 2 * H:4 * H] = tq

    part = pl.BlockSpec((2, _T_NP, H), lambda i: (0, i, 0))
    single = pl.BlockSpec((_T_NP, H), lambda i: (i, 0))
    dbl = pl.BlockSpec((_T_NP, 2 * H), lambda i: (i, 0))
    wspec = pl.BlockSpec((5 * H, 2 * H), lambda i: (0, 0))
    return pl.pallas_call(
        body,
        grid=(NNP // _T_NP,),
        in_specs=[part, part, part, single, single, dbl, dbl,
                  wspec, wspec],
        out_specs=pl.BlockSpec((_T_NP, 4 * H), lambda i: (i, 0)),
        out_shape=jax.ShapeDtypeStruct((NNP, 4 * H), F32),
    )(s2a, s2b, sr, cnt5, cnt6, t5c, t6c, wb, wc)


def _tc_tbl(parts, cnt, nse):
    def body(p_ref, c_ref, n_ref, o_ref):
        o_ref[:, 0:H] = p_ref[0] + p_ref[1]
        o_ref[:, H:2 * H] = c_ref[...] * n_ref[...]

    single = pl.BlockSpec((_T_NP, H), lambda i: (i, 0))
    return pl.pallas_call(
        body,
        grid=(NNP // _T_NP,),
        in_specs=[pl.BlockSpec((2, _T_NP, H), lambda i: (0, i, 0)),
                  single, single],
        out_specs=pl.BlockSpec((_T_NP, 2 * H), lambda i: (i, 0)),
        out_shape=jax.ShapeDtypeStruct((NNP, 2 * H), F32),
    )(parts, cnt, nse)


def _tc_mul2(x, y):
    def body(x_ref, y_ref, o_ref):
        o_ref[...] = x_ref[...] * y_ref[...]

    spec = pl.BlockSpec((_T_NP, H), lambda i: (i, 0))
    return pl.pallas_call(
        body,
        grid=(NNP // _T_NP,),
        in_specs=[spec, spec],
        out_specs=spec,
        out_shape=jax.ShapeDtypeStruct((NNP, H), F32),
    )(x, y)


def _tc_edge_pass1(er, g12, wa, *, rows, row_off):
    tile = 640

    def body(er_ref, g12_ref, wa_ref, y_ref, st_ref):
        i = pl.program_id(0)
        g1 = g12_ref[:, 0:2 * H]
        g2 = g12_ref[:, 2 * H:4 * H]
        up = pltpu.roll(g2, tile - 1, 0)
        dn = pltpu.roll(g2, 1, 0)
        parity = lax.broadcasted_iota(jnp.int32, (tile, 2 * H), 0) % 2
        g2s = jnp.where(parity == 0, up, dn)
        y = (jnp.dot(er_ref[...], wa_ref[...], preferred_element_type=F32)
             + g1 + g2s)
        y_ref[...] = y

        @pl.when(i == 0)
        def _():
            st_ref[...] = jnp.zeros_like(st_ref)

        st_ref[0:1, :] += jnp.sum(y, axis=0, keepdims=True)
        st_ref[1:2, :] += jnp.sum(y * y, axis=0, keepdims=True)

    return pl.pallas_call(
        body,
        grid=(rows // tile,),
        in_specs=[
            pl.BlockSpec((tile, H), lambda i: (i + row_off, 0)),
            pl.BlockSpec((tile, 4 * H), lambda i: (i, 0)),
            pl.BlockSpec((H, 2 * H), lambda i: (0, 0)),
        ],
        out_specs=[
            pl.BlockSpec((tile, 2 * H), lambda i: (i, 0)),
            pl.BlockSpec((8, 2 * H), lambda i: (0, 0)),
        ],
        out_shape=[
            jax.ShapeDtypeStruct((rows, 2 * H), F32),
            jax.ShapeDtypeStruct((8, 2 * H), F32),
        ],
    )(er, g12, wa)


def _tc_cycle_pass1(p2c, r2a, r2b, crep, vc, vba, vbb, vr, *,
                    rows, row_off):
    tile = 400

    def body(p2_ref, r2a_ref, r2b_ref, cr_ref,
             vc_ref, vba_ref, vbb_ref, vr_ref, y_ref, st_ref):
        i = pl.program_id(0)
        y = (jnp.dot(p2_ref[...], vc_ref[...], preferred_element_type=F32)
             + jnp.dot(r2a_ref[...], vba_ref[...], preferred_element_type=F32)
             + jnp.dot(r2b_ref[...], vbb_ref[...], preferred_element_type=F32)
             + jnp.dot(cr_ref[...], vr_ref[...], preferred_element_type=F32))
        y_ref[...] = y

        @pl.when(i == 0)
        def _():
            st_ref[...] = jnp.zeros_like(st_ref)

        st_ref[0:1, :] += jnp.sum(y, axis=0, keepdims=True)
        st_ref[1:2, :] += jnp.sum(y * y, axis=0, keepdims=True)

    row = pl.BlockSpec((tile, H), lambda i: (i, 0))
    dblrow = pl.BlockSpec((tile, 2 * H), lambda i: (i, 0))
    woff = pl.BlockSpec((tile, H), lambda i: (i + row_off, 0))
    wmat = pl.BlockSpec((H, 2 * H), lambda i: (0, 0))
    wdbl = pl.BlockSpec((2 * H, 2 * H), lambda i: (0, 0))
    return pl.pallas_call(
        body,
        grid=(rows // tile,),
        in_specs=[dblrow, row, row, woff, wdbl, wmat, wmat, wmat],
        out_specs=[
            pl.BlockSpec((tile, 2 * H), lambda i: (i, 0)),
            pl.BlockSpec((8, 2 * H), lambda i: (0, 0)),
        ],
        out_shape=[
            jax.ShapeDtypeStruct((rows, 2 * H), F32),
            jax.ShapeDtypeStruct((8, 2 * H), F32),
        ],
    )(p2c, r2a, r2b, crep, vc, vba, vbb, vr)


def _tc_pass2(y, st, w2, gg, bb, *, n_total, tile, rows):
    inv_n = 1.0 / float(n_total)

    def body(y_ref, st_ref, w2_ref, g_ref, b_ref, z_ref, st2_ref):
        i = pl.program_id(0)
        m = st_ref[0:1, :] * inv_n
        v = st_ref[1:2, :] * inv_n - m * m
        r = lax.rsqrt(v + 1e-5)
        h = jnp.maximum((y_ref[...] - m) * r * g_ref[...] + b_ref[...], 0.0)
        z = jnp.dot(h, w2_ref[...], preferred_element_type=F32)
        z_ref[...] = z

        @pl.when(i == 0)
        def _():
            st2_ref[...] = jnp.zeros_like(st2_ref)

        st2_ref[0:1, :] += jnp.sum(z, axis=0, keepdims=True)
        st2_ref[1:2, :] += jnp.sum(z * z, axis=0, keepdims=True)

    return pl.pallas_call(
        body,
        grid=(rows // tile,),
        in_specs=[
            pl.BlockSpec((tile, 2 * H), lambda i: (i, 0)),
            pl.BlockSpec((8, 2 * H), lambda i: (0, 0)),
            pl.BlockSpec((2 * H, H), lambda i: (0, 0)),
            pl.BlockSpec((1, 2 * H), lambda i: (0, 0)),
            pl.BlockSpec((1, 2 * H), lambda i: (0, 0)),
        ],
        out_specs=[
            pl.BlockSpec((tile, H), lambda i: (i, 0)),
            pl.BlockSpec((8, H), lambda i: (0, 0)),
        ],
        out_shape=[
            jax.ShapeDtypeStruct((rows, H), F32),
            jax.ShapeDtypeStruct((8, H), F32),
        ],
    )(y, st, w2, gg, bb)


def _tc_pass3(z, st2, gg, bb, *, n_total, tile, rows):
    inv_n = 1.0 / float(n_total)

    def body(z_ref, st_ref, g_ref, b_ref, o_ref):
        m = st_ref[0:1, :] * inv_n
        v = st_ref[1:2, :] * inv_n - m * m
        r = lax.rsqrt(v + 1e-5)
        o_ref[...] = jnp.maximum(
            (z_ref[...] - m) * r * g_ref[...] + b_ref[...], 0.0)

    return pl.pallas_call(
        body,
        grid=(rows // tile,),
        in_specs=[
            pl.BlockSpec((tile, H), lambda i: (i, 0)),
            pl.BlockSpec((8, H), lambda i: (0, 0)),
            pl.BlockSpec((1, H), lambda i: (0, 0)),
            pl.BlockSpec((1, H), lambda i: (0, 0)),
        ],
        out_specs=pl.BlockSpec((tile, H), lambda i: (i, 0)),
        out_shape=jax.ShapeDtypeStruct((rows, H), F32),
    )(z, st2, gg, bb)


def kernel(edge_rep, cycle_rep, edge_nodes, cycle5_nodes, cycle6_nodes,
           eW1, eg1, eb1, eW2, eg2, eb2, cW1, cg1, cb1, cW2, cg2, cb2):
    en = edge_nodes.astype(jnp.int32)
    c5 = cycle5_nodes.astype(jnp.int32)
    c6 = cycle6_nodes.astype(jnp.int32)

    padv = jnp.int32(NN)
    c5p = jnp.concatenate([c5, jnp.full((B5 - A5,), padv)])
    c6p = jnp.concatenate([c6, jnp.full((B6 - A6,), padv)])
    enp = jnp.concatenate([en, jnp.full((BEG - AE,), padv)])
    ens = en.reshape(E, 2)[:, ::-1].reshape(AE)
    ensp = jnp.concatenate([ens, jnp.full((BEG - AE,), padv)])
    c6shift = jnp.concatenate([jnp.full((B6 - A6,), padv), c6])

    zinit = jnp.zeros((NC, NNP, H), F32)

    ones_rows = jnp.ones((max(B5, B6), H), F32)

    (sc1_p,) = _sc_scatter_multi([[(edge_rep, en, AE, 0)]], zinit, chunk=80)
    ns_e = _tc_add2(sc1_p)
    (cnt5_p,) = _sc_scatter_multi([[(ones_rows, c5p, B5, 0)]], zinit,
                                  chunk=128)
    (cnt6_p,) = _sc_scatter_multi([[(ones_rows, c6p, B6, 0)]], zinit,
                                  chunk=128)
    cnt5 = _tc_add2(cnt5_p)
    cnt6 = _tc_add2(cnt6_p)
    (g5,) = _sc_gather_multi([(ns_e, c5p, B5)], chunk=256, width=H)
    (g6,) = _sc_gather_multi([(ns_e, c6p, B6)], chunk=256, width=H)
    r1_5 = _tc_gsum_rep(g5, g=5, rows=A5, tile=400, out_rows=B5)
    r1_6 = _tc_gsum_rep(g6, g=6, rows=A6, tile=480, out_rows=B6)

    (t5a_p,) = _sc_scatter_multi([[(r1_5, c5p, B5, 0)]], zinit, chunk=128)
    (t6a_p,) = _sc_scatter_multi([[(r1_6, c6p, B6, 0)]], zinit, chunk=128)
    t5c = _tc_tbl(t5a_p, cnt5, ns_e)
    t6c = _tc_tbl(t6a_p, cnt6, ns_e)

    (p2_5,) = _sc_gather_multi([(t5c, c5p, B5)], chunk=128, width=2 * H)
    (p2_6,) = _sc_gather_multi([(t6c, c6p, B6)], chunk=128, width=2 * H)

    r2_5a, r2_5b = _tc_gsum_rep2(p2_5, g=5, rows=A5, tile=400, out_rows=B5)
    r2_6a, r2_6b = _tc_gsum_rep2(p2_6, g=6, rows=A6, tile=480, out_rows=B6)

    (s2a,) = _sc_scatter_multi(
        [[(r2_5a, c5p, B5, 0), (r2_6a, c6p, B6, 0)]], zinit, chunk=128)
    (s2b,) = _sc_scatter_multi(
        [[(r2_5b, c5p, B5, 0), (r2_6b, c6p, B6, 0)]], zinit, chunk=128)
    (sr,) = _sc_scatter_multi(
        [[(cycle_rep, c5p, B5, 0), (cycle_rep, c6shift, B6, AC - B6)]],
        zinit, chunk=128)

    wb = eW1[H:6 * H, :]
    wc = eW1[6 * H:, :]
    t12 = _tc_te(s2a, s2b, sr, cnt5, cnt6, t5c, t6c, wb, wc)

    BH = BEG // 2
    RH1 = BH
    RH2 = AE - BH
    wa = eW1[:H, :]
    eg1r = eg1.reshape(1, -1)
    eb1r = eb1.reshape(1, -1)
    eg2r = eg2.reshape(1, -1)
    eb2r = eb2.reshape(1, -1)
    enp_h2 = enp[BH:]
    (g12_h1,) = _sc_gather_multi([(t12, en, BH)], chunk=80, width=4 * H)
    y1, st_a = _tc_edge_pass1(edge_rep, g12_h1, wa, rows=RH1, row_off=0)
    (g12_h2,) = _sc_gather_multi([(t12, enp_h2, BH)], chunk=80,
                                 width=4 * H)
    y2, st_b = _tc_edge_pass1(edge_rep, g12_h2, wa,
                              rows=RH2, row_off=RH1 // 640)
    st_e = st_a + st_b
    z1, st2_a = _tc_pass2(y1, st_e, eW2, eg1r, eb1r,
                          n_total=AE, tile=640, rows=RH1)
    z2, st2_b = _tc_pass2(y2, st_e, eW2, eg1r, eb1r,
                          n_total=AE, tile=640, rows=RH2)
    st2_e = st2_a + st2_b
    eo1 = _tc_pass3(z1, st2_e, eg2r, eb2r, n_total=AE, tile=640, rows=RH1)
    eo2 = _tc_pass3(z2, st2_e, eg2r, eb2r, n_total=AE, tile=640, rows=RH2)
    edge_out = jnp.concatenate([eo1, eo2], axis=0)

    vb_a = cW1[0:H, :]
    vb_b = cW1[H:2 * H, :]
    vc = cW1[2 * H:4 * H, :]
    vr = cW1[4 * H:, :]
    y5, st5 = _tc_cycle_pass1(p2_5, r2_5a, r2_5b, cycle_rep,
                              vc, vb_a, vb_b, vr, rows=A5, row_off=0)
    y6, st6 = _tc_cycle_pass1(p2_6, r2_6a, r2_6b, cycle_rep,
                              vc, vb_a, vb_b, vr,
                              rows=A6, row_off=A5 // 400)
    st_c = st5 + st6
    cg1r = cg1.reshape(1, -1)
    cb1r = cb1.reshape(1, -1)
    z5, st2_5 = _tc_pass2(y5, st_c, cW2, cg1r, cb1r,
                          n_total=AC, tile=400, rows=A5)
    z6, st2_6 = _tc_pass2(y6, st_c, cW2, cg1r, cb1r,
                          n_total=AC, tile=400, rows=A6)
    st2_c = st2_5 + st2_6
    cg2r = cg2.reshape(1, -1)
    cb2r = cb2.reshape(1, -1)
    co5 = _tc_pass3(z5, st2_c, cg2r, cb2r, n_total=AC, tile=400, rows=A5)
    co6 = _tc_pass3(z6, st2_c, cg2r, cb2r, n_total=AC, tile=400, rows=A6)
    cycle_out = jnp.concatenate([co5, co6], axis=0)

    return edge_out, cycle_out


# speedup vs baseline: 1.9048x; 1.0001x over previous
import jax
import jax.numpy as jnp
from jax import lax
from jax.experimental import pallas as pl
from jax.experimental.pallas import tpu as pltpu
from jax.experimental.pallas import tpu_sc as plsc

H = 128
NN = 10000
E = 160000
AE = 2 * E
C5 = 10000
C6 = 10000
A5 = 5 * C5
A6 = 6 * C6
AC = A5 + A6
NNP = NN + 8
NC, NS = 2, 16
NW = NC * NS
B5 = 57344
B6 = 65536
BEG = 327680
F32 = jnp.float32


def _sc_scatter_multi(tables, zinit, *, chunk):
    arrays = [zinit]
    index_of = {id(zinit): 0}
    plan = []
    for stages in tables:
        sp = []
        for (a, idx, btot, base) in stages:
            for arr in (a, idx):
                if id(arr) not in index_of:
                    index_of[id(arr)] = len(arrays)
                    arrays.append(arr)
            share = btot // NW
            assert share % chunk == 0
            sp.append((index_of[id(a)], index_of[id(idx)], share, base))
        plan.append(sp)
    n_in = len(arrays)
    n_tbl = len(tables)
    mesh = plsc.VectorSubcoreMesh(core_axis_name="c", subcore_axis_name="s")

    def body(*refs):
        ins = refs[:n_in]
        outs = refs[n_in:n_in + n_tbl]
        i0, i1, r0, r1, table, l0, l1, s0, s1 = refs[n_in + n_tbl:]
        idx_v = (i0, i1)
        rows_v = (r0, r1)
        lsem = (l0, l1)
        ssem = (s0, s1)
        c = lax.axis_index("c")
        s = lax.axis_index("s")
        wid = s * NC + c

        for k, sp in enumerate(plan):
            @pl.when(s == 0)
            def _():
                pltpu.sync_copy(ins[0].at[c], table)

            plsc.subcore_barrier()
            for (ai, ii, share, src_base) in sp:
                src_ref = ins[ai]
                idx_ref = ins[ii]
                base = wid * share
                nit = share // chunk

                def step(jj, carry, src_ref=src_ref, idx_ref=idx_ref,
                         base=base, src_base=src_base, nit=nit):
                    for b in range(2):
                        off = base + (2 * jj + b) * chunk

                        @pl.when(jj > 0)
                        def _():
                            pltpu.make_async_copy(
                                rows_v[b], table.at[idx_v[b]],
                                ssem[b]).wait()

                        pltpu.sync_copy(idx_ref.at[pl.ds(off, chunk)],
                                        idx_v[b])
                        pltpu.async_copy(
                            src_ref.at[pl.ds(src_base + off, chunk)],
                            rows_v[b], lsem[b])
                    for b in range(2):
                        pltpu.make_async_copy(
                            src_ref.at[pl.ds(src_base + base, chunk)],
                            rows_v[b], lsem[b]).wait()
                        pltpu.async_copy(rows_v[b], table.at[idx_v[b]],
                                         ssem[b], add=True)
                    return carry

                lax.fori_loop(0, nit // 2, step, 0)
                if nit % 2 == 1:
                    off = base + (nit - 1) * chunk
                    pltpu.make_async_copy(
                        rows_v[0], table.at[idx_v[0]], ssem[0]).wait()
                    pltpu.sync_copy(idx_ref.at[pl.ds(off, chunk)], idx_v[0])
                    pltpu.sync_copy(
                        src_ref.at[pl.ds(src_base + off, chunk)], rows_v[0])
                    pltpu.async_copy(rows_v[0], table.at[idx_v[0]], ssem[0],
                                     add=True)
                for b in range(2):
                    pltpu.make_async_copy(
                        rows_v[b], table.at[idx_v[b]], ssem[b]).wait()
            plsc.subcore_barrier()

            @pl.when(s == 0)
            def _():
                pltpu.sync_copy(table, outs[k].at[c])

            if k + 1 < n_tbl:
                plsc.subcore_barrier()

    f = pl.kernel(
        body,
        out_type=[jax.ShapeDtypeStruct((NC, NNP, H), F32)] * n_tbl,
        mesh=mesh,
        scratch_types=[
            pltpu.VMEM((chunk,), jnp.int32),
            pltpu.VMEM((chunk,), jnp.int32),
            pltpu.VMEM((chunk, H), F32),
            pltpu.VMEM((chunk, H), F32),
            pltpu.VMEM_SHARED((NNP, H), F32),
            pltpu.SemaphoreType.DMA,
            pltpu.SemaphoreType.DMA,
            pltpu.SemaphoreType.DMA,
            pltpu.SemaphoreType.DMA,
        ],
    )
    return f(*arrays)


def _sc_gather_multi(stages, *, chunk, width, dtype=F32):
    arrays = []
    index_of = {}
    plan = []
    for (t, idx, btot) in stages:
        for arr in (t, idx):
            if id(arr) not in index_of:
                index_of[id(arr)] = len(arrays)
                arrays.append(arr)
        share = btot // NW
        assert share % chunk == 0
        plan.append((index_of[id(t)], index_of[id(idx)], share, btot))
    n_in = len(arrays)
    n_st = len(stages)
    mesh = plsc.VectorSubcoreMesh(core_axis_name="c", subcore_axis_name="s")

    def body(*refs):
        ins = refs[:n_in]
        outs = refs[n_in:n_in + n_st]
        i0, i1, r0, r1, g0, g1, s0, s1 = refs[n_in + n_st:]
        idx_v = (i0, i1)
        rows_v = (r0, r1)
        gsem = (g0, g1)
        ssem = (s0, s1)
        c = lax.axis_index("c")
        s = lax.axis_index("s")
        wid = s * NC + c

        for k, (ti, ii, share, _) in enumerate(plan):
            table_ref = ins[ti]
            idx_ref = ins[ii]
            out_ref = outs[k]
            base = wid * share
            nit = share // chunk

            def step(jj, carry, table_ref=table_ref, idx_ref=idx_ref,
                     out_ref=out_ref, base=base):
                for b in range(2):
                    off = base + (2 * jj + b) * chunk

                    @pl.when(jj > 0)
                    def _():
                        pltpu.make_async_copy(
                            rows_v[b], out_ref.at[pl.ds(base, chunk)],
                            ssem[b]).wait()

                    pltpu.sync_copy(idx_ref.at[pl.ds(off, chunk)], idx_v[b])
                    pltpu.async_copy(table_ref.at[idx_v[b]], rows_v[b],
                                     gsem[b])
                for b in range(2):
                    off = base + (2 * jj + b) * chunk
                    pltpu.make_async_copy(table_ref.at[idx_v[b]],
                                          rows_v[b], gsem[b]).wait()
                    pltpu.async_copy(rows_v[b],
                                     out_ref.at[pl.ds(off, chunk)], ssem[b])
                return carry

            lax.fori_loop(0, nit // 2, step, 0)
            if nit % 2 == 1:
                off = base + (nit - 1) * chunk
                pltpu.make_async_copy(
                    rows_v[0], out_ref.at[pl.ds(base, chunk)], ssem[0]).wait()
                pltpu.sync_copy(idx_ref.at[pl.ds(off, chunk)], idx_v[0])
                pltpu.async_copy(table_ref.at[idx_v[0]], rows_v[0], gsem[0])
                pltpu.make_async_copy(table_ref.at[idx_v[0]], rows_v[0],
                                      gsem[0]).wait()
                pltpu.async_copy(rows_v[0], out_ref.at[pl.ds(off, chunk)],
                                 ssem[0])
            for b in range(2):
                pltpu.make_async_copy(
                    rows_v[b], out_ref.at[pl.ds(base, chunk)], ssem[b]).wait()

    f = pl.kernel(
        body,
        out_type=[jax.ShapeDtypeStruct((btot, width), dtype)
                  for (_, _, btot) in stages],
        mesh=mesh,
        scratch_types=[
            pltpu.VMEM((chunk,), jnp.int32),
            pltpu.VMEM((chunk,), jnp.int32),
            pltpu.VMEM((chunk, width), dtype),
            pltpu.VMEM((chunk, width), dtype),
            pltpu.SemaphoreType.DMA,
            pltpu.SemaphoreType.DMA,
            pltpu.SemaphoreType.DMA,
            pltpu.SemaphoreType.DMA,
        ],
    )
    return f(*arrays)


_T_NP = 72


def _tc_add2(parts):
    def body(p_ref, o_ref):
        o_ref[...] = p_ref[0] + p_ref[1]

    return pl.pallas_call(
        body,
        grid=(NNP // _T_NP,),
        in_specs=[pl.BlockSpec((2, _T_NP, H), lambda i: (0, i, 0))],
        out_specs=pl.BlockSpec((_T_NP, H), lambda i: (i, 0)),
        out_shape=jax.ShapeDtypeStruct((NNP, H), F32),
    )(parts)


def _tc_gsum_rep2(x, *, g, rows, tile, out_rows):
    def body(x_ref, o1_ref, o2_ref):
        ri = lax.broadcasted_iota(jnp.int32, (tile, tile), 0) // g
        ci = lax.broadcasted_iota(jnp.int32, (tile, tile), 1) // g
        bmat = (ri == ci).astype(F32)
        rep = jnp.dot(bmat, x_ref[...], preferred_element_type=F32)
        o1_ref[...] = rep[:, 0:H]
        o2_ref[...] = rep[:, H:2 * H]

    return pl.pallas_call(
        body,
        grid=(rows // tile,),
        in_specs=[pl.BlockSpec((tile, 2 * H), lambda i: (i, 0))],
        out_specs=[pl.BlockSpec((tile, H), lambda i: (i, 0))] * 2,
        out_shape=[jax.ShapeDtypeStruct((out_rows, H), F32)] * 2,
    )(x)


def _tc_gsum_rep(x, *, g, rows, tile, out_rows):
    def body(x_ref, o_ref):
        ri = lax.broadcasted_iota(jnp.int32, (tile, tile), 0) // g
        ci = lax.broadcasted_iota(jnp.int32, (tile, tile), 1) // g
        bmat = (ri == ci).astype(F32)
        o_ref[...] = jnp.dot(bmat, x_ref[...], preferred_element_type=F32)

    return pl.pallas_call(
        body,
        grid=(rows // tile,),
        in_specs=[pl.BlockSpec((tile, H), lambda i: (i, 0))],
        out_specs=pl.BlockSpec((tile, H), lambda i: (i, 0)),
        out_shape=jax.ShapeDtypeStruct((out_rows, H), F32),
    )(x)


def _tc_te(s2a, s2b, sr, cnt5, cnt6, t5c, t6c, wb, wc):
    def body(a_ref, b_ref, e_ref, c5_ref, c6_ref, t5_ref, t6_ref,
             wb_ref, wc_ref, t12_ref):
        pieces = (
            a_ref[0] + a_ref[1],
            b_ref[0] + b_ref[1],
            c5_ref[...] * t5_ref[:, 0:H] + c6_ref[...] * t6_ref[:, 0:H],
            c5_ref[...] * t5_ref[:, H:2 * H]
            + c6_ref[...] * t6_ref[:, H:2 * H],
            e_ref[0] + e_ref[1],
        )
        t1 = jnp.zeros((_T_NP, 2 * H), F32)
        tq = jnp.zeros((_T_NP, 2 * H), F32)
        for k, piece in enumerate(pieces):
            wbk = wb_ref[k * H:(k + 1) * H, :]
            wck = wc_ref[k * H:(k + 1) * H, :]
            t1 = t1 + jnp.dot(piece, wbk + wck, preferred_element_type=F32)
            tq = tq + jnp.dot(piece, wbk, preferred_element_type=F32)
        t12_ref[:, 0:2 * H] = t1
        t12_ref[:, 2 * H:4 * H] = tq

    part = pl.BlockSpec((2, _T_NP, H), lambda i: (0, i, 0))
    single = pl.BlockSpec((_T_NP, H), lambda i: (i, 0))
    dbl = pl.BlockSpec((_T_NP, 2 * H), lambda i: (i, 0))
    wspec = pl.BlockSpec((5 * H, 2 * H), lambda i: (0, 0))
    return pl.pallas_call(
        body,
        grid=(NNP // _T_NP,),
        in_specs=[part, part, part, single, single, dbl, dbl,
                  wspec, wspec],
        out_specs=pl.BlockSpec((_T_NP, 4 * H), lambda i: (i, 0)),
        out_shape=jax.ShapeDtypeStruct((NNP, 4 * H), F32),
    )(s2a, s2b, sr, cnt5, cnt6, t5c, t6c, wb, wc)


def _tc_tbl(parts, cnt, nse):
    def body(p_ref, c_ref, n_ref, o_ref):
        o_ref[:, 0:H] = p_ref[0] + p_ref[1]
        o_ref[:, H:2 * H] = c_ref[...] * n_ref[...]

    single = pl.BlockSpec((_T_NP, H), lambda i: (i, 0))
    return pl.pallas_call(
        body,
        grid=(NNP // _T_NP,),
        in_specs=[pl.BlockSpec((2, _T_NP, H), lambda i: (0, i, 0)),
                  single, single],
        out_specs=pl.BlockSpec((_T_NP, 2 * H), lambda i: (i, 0)),
        out_shape=jax.ShapeDtypeStruct((NNP, 2 * H), F32),
    )(parts, cnt, nse)


def _tc_edge_pass1(er, g12, wa, *, rows, row_off):
    tile = 640

    def body(er_ref, g12_ref, wa_ref, y_ref, st_ref):
        i = pl.program_id(0)
        g1 = g12_ref[:, 0:2 * H]
        g2 = g12_ref[:, 2 * H:4 * H]
        up = pltpu.roll(g2, tile - 1, 0)
        dn = pltpu.roll(g2, 1, 0)
        parity = lax.broadcasted_iota(jnp.int32, (tile, 2 * H), 0) % 2
        g2s = jnp.where(parity == 0, up, dn)
        y = (jnp.dot(er_ref[...], wa_ref[...], preferred_element_type=F32)
             + g1 + g2s)
        y_ref[...] = y

        @pl.when(i == 0)
        def _():
            st_ref[...] = jnp.zeros_like(st_ref)

        st_ref[0:1, :] += jnp.sum(y, axis=0, keepdims=True)
        st_ref[1:2, :] += jnp.sum(y * y, axis=0, keepdims=True)

    return pl.pallas_call(
        body,
        grid=(rows // tile,),
        in_specs=[
            pl.BlockSpec((tile, H), lambda i: (i + row_off, 0)),
            pl.BlockSpec((tile, 4 * H), lambda i: (i, 0)),
            pl.BlockSpec((H, 2 * H), lambda i: (0, 0)),
        ],
        out_specs=[
            pl.BlockSpec((tile, 2 * H), lambda i: (i, 0)),
            pl.BlockSpec((8, 2 * H), lambda i: (0, 0)),
        ],
        out_shape=[
            jax.ShapeDtypeStruct((rows, 2 * H), F32),
            jax.ShapeDtypeStruct((8, 2 * H), F32),
        ],
    )(er, g12, wa)


def _tc_cycle_pass1(p2c, r2a, r2b, crep, vc, vba, vbb, vr, *,
                    rows, row_off):
    tile = 400

    def body(p2_ref, r2a_ref, r2b_ref, cr_ref,
             vc_ref, vba_ref, vbb_ref, vr_ref, y_ref, st_ref):
        i = pl.program_id(0)
        y = (jnp.dot(p2_ref[...], vc_ref[...], preferred_element_type=F32)
             + jnp.dot(r2a_ref[...], vba_ref[...], preferred_element_type=F32)
             + jnp.dot(r2b_ref[...], vbb_ref[...], preferred_element_type=F32)
             + jnp.dot(cr_ref[...], vr_ref[...], preferred_element_type=F32))
        y_ref[...] = y

        @pl.when(i == 0)
        def _():
            st_ref[...] = jnp.zeros_like(st_ref)

        st_ref[0:1, :] += jnp.sum(y, axis=0, keepdims=True)
        st_ref[1:2, :] += jnp.sum(y * y, axis=0, keepdims=True)

    row = pl.BlockSpec((tile, H), lambda i: (i, 0))
    dblrow = pl.BlockSpec((tile, 2 * H), lambda i: (i, 0))
    woff = pl.BlockSpec((tile, H), lambda i: (i + row_off, 0))
    wmat = pl.BlockSpec((H, 2 * H), lambda i: (0, 0))
    wdbl = pl.BlockSpec((2 * H, 2 * H), lambda i: (0, 0))
    return pl.pallas_call(
        body,
        grid=(rows // tile,),
        in_specs=[dblrow, row, row, woff, wdbl, wmat, wmat, wmat],
        out_specs=[
            pl.BlockSpec((tile, 2 * H), lambda i: (i, 0)),
            pl.BlockSpec((8, 2 * H), lambda i: (0, 0)),
        ],
        out_shape=[
            jax.ShapeDtypeStruct((rows, 2 * H), F32),
            jax.ShapeDtypeStruct((8, 2 * H), F32),
        ],
    )(p2c, r2a, r2b, crep, vc, vba, vbb, vr)


def _tc_pass2(y, st, w2, gg, bb, *, n_total, tile, rows):
    inv_n = 1.0 / float(n_total)

    def body(y_ref, st_ref, w2_ref, g_ref, b_ref, z_ref, st2_ref):
        i = pl.program_id(0)
        m = st_ref[0:1, :] * inv_n
        v = st_ref[1:2, :] * inv_n - m * m
        r = lax.rsqrt(v + 1e-5)
        h = jnp.maximum((y_ref[...] - m) * r * g_ref[...] + b_ref[...], 0.0)
        z = jnp.dot(h, w2_ref[...], preferred_element_type=F32)
        z_ref[...] = z

        @pl.when(i == 0)
        def _():
            st2_ref[...] = jnp.zeros_like(st2_ref)

        st2_ref[0:1, :] += jnp.sum(z, axis=0, keepdims=True)
        st2_ref[1:2, :] += jnp.sum(z * z, axis=0, keepdims=True)

    return pl.pallas_call(
        body,
        grid=(rows // tile,),
        in_specs=[
            pl.BlockSpec((tile, 2 * H), lambda i: (i, 0)),
            pl.BlockSpec((8, 2 * H), lambda i: (0, 0)),
            pl.BlockSpec((2 * H, H), lambda i: (0, 0)),
            pl.BlockSpec((1, 2 * H), lambda i: (0, 0)),
            pl.BlockSpec((1, 2 * H), lambda i: (0, 0)),
        ],
        out_specs=[
            pl.BlockSpec((tile, H), lambda i: (i, 0)),
            pl.BlockSpec((8, H), lambda i: (0, 0)),
        ],
        out_shape=[
            jax.ShapeDtypeStruct((rows, H), F32),
            jax.ShapeDtypeStruct((8, H), F32),
        ],
    )(y, st, w2, gg, bb)


def _tc_pass3(z, st2, gg, bb, *, n_total, tile, rows):
    inv_n = 1.0 / float(n_total)

    def body(z_ref, st_ref, g_ref, b_ref, o_ref):
        m = st_ref[0:1, :] * inv_n
        v = st_ref[1:2, :] * inv_n - m * m
        r = lax.rsqrt(v + 1e-5)
        o_ref[...] = jnp.maximum(
            (z_ref[...] - m) * r * g_ref[...] + b_ref[...], 0.0)

    return pl.pallas_call(
        body,
        grid=(rows // tile,),
        in_specs=[
            pl.BlockSpec((tile, H), lambda i: (i, 0)),
            pl.BlockSpec((8, H), lambda i: (0, 0)),
            pl.BlockSpec((1, H), lambda i: (0, 0)),
            pl.BlockSpec((1, H), lambda i: (0, 0)),
        ],
        out_specs=pl.BlockSpec((tile, H), lambda i: (i, 0)),
        out_shape=jax.ShapeDtypeStruct((rows, H), F32),
    )(z, st2, gg, bb)


def kernel(edge_rep, cycle_rep, edge_nodes, cycle5_nodes, cycle6_nodes,
           eW1, eg1, eb1, eW2, eg2, eb2, cW1, cg1, cb1, cW2, cg2, cb2):
    en = edge_nodes.astype(jnp.int32)
    c5 = cycle5_nodes.astype(jnp.int32)
    c6 = cycle6_nodes.astype(jnp.int32)

    padv = jnp.int32(NN)
    c5p = jnp.concatenate([c5, jnp.full((B5 - A5,), padv)])
    c6p = jnp.concatenate([c6, jnp.full((B6 - A6,), padv)])
    enp = jnp.concatenate([en, jnp.full((BEG - AE,), padv)])
    c6shift = jnp.concatenate([jnp.full((B6 - A6,), padv), c6])

    zinit = jnp.zeros((NC, NNP, H), F32)

    ones_rows = jnp.ones((max(B5, B6), H), F32)

    (sc1_p,) = _sc_scatter_multi([[(edge_rep, en, AE, 0)]], zinit, chunk=80)
    ns_e = _tc_add2(sc1_p)
    (cnt5_p,) = _sc_scatter_multi([[(ones_rows, c5p, B5, 0)]], zinit,
                                  chunk=128)
    (cnt6_p,) = _sc_scatter_multi([[(ones_rows, c6p, B6, 0)]], zinit,
                                  chunk=128)
    cnt5 = _tc_add2(cnt5_p)
    cnt6 = _tc_add2(cnt6_p)
    (g5,) = _sc_gather_multi([(ns_e, c5p, B5)], chunk=256, width=H)
    (g6,) = _sc_gather_multi([(ns_e, c6p, B6)], chunk=256, width=H)
    r1_5 = _tc_gsum_rep(g5, g=5, rows=A5, tile=400, out_rows=B5)
    r1_6 = _tc_gsum_rep(g6, g=6, rows=A6, tile=480, out_rows=B6)

    (t5a_p,) = _sc_scatter_multi([[(r1_5, c5p, B5, 0)]], zinit, chunk=128)
    (t6a_p,) = _sc_scatter_multi([[(r1_6, c6p, B6, 0)]], zinit, chunk=128)
    t5c = _tc_tbl(t5a_p, cnt5, ns_e)
    t6c = _tc_tbl(t6a_p, cnt6, ns_e)

    (p2_5,) = _sc_gather_multi([(t5c, c5p, B5)], chunk=128, width=2 * H)
    (p2_6,) = _sc_gather_multi([(t6c, c6p, B6)], chunk=128, width=2 * H)

    r2_5a, r2_5b = _tc_gsum_rep2(p2_5, g=5, rows=A5, tile=400, out_rows=B5)
    r2_6a, r2_6b = _tc_gsum_rep2(p2_6, g=6, rows=A6, tile=480, out_rows=B6)

    (s2a,) = _sc_scatter_multi(
        [[(r2_5a, c5p, B5, 0), (r2_6a, c6p, B6, 0)]], zinit, chunk=128)
    (s2b,) = _sc_scatter_multi(
        [[(r2_5b, c5p, B5, 0), (r2_6b, c6p, B6, 0)]], zinit, chunk=128)
    (sr,) = _sc_scatter_multi(
        [[(cycle_rep, c5p, B5, 0), (cycle_rep, c6shift, B6, AC - B6)]],
        zinit, chunk=128)

    wb = eW1[H:6 * H, :]
    wc = eW1[6 * H:, :]
    t12 = _tc_te(s2a, s2b, sr, cnt5, cnt6, t5c, t6c, wb, wc)

    BH = BEG // 2
    RH1 = BH
    RH2 = AE - BH
    wa = eW1[:H, :]
    eg1r = eg1.reshape(1, -1)
    eb1r = eb1.reshape(1, -1)
    eg2r = eg2.reshape(1, -1)
    eb2r = eb2.reshape(1, -1)
    enp_h2 = enp[BH:]
    (g12_h1,) = _sc_gather_multi([(t12, en, BH)], chunk=80, width=4 * H)
    y1, st_a = _tc_edge_pass1(edge_rep, g12_h1, wa, rows=RH1, row_off=0)
    (g12_h2,) = _sc_gather_multi([(t12, enp_h2, BH)], chunk=80,
                                 width=4 * H)
    y2, st_b = _tc_edge_pass1(edge_rep, g12_h2, wa,
                              rows=RH2, row_off=RH1 // 640)
    st_e = st_a + st_b
    z1, st2_a = _tc_pass2(y1, st_e, eW2, eg1r, eb1r,
                          n_total=AE, tile=640, rows=RH1)
    z2, st2_b = _tc_pass2(y2, st_e, eW2, eg1r, eb1r,
                          n_total=AE, tile=640, rows=RH2)
    st2_e = st2_a + st2_b
    eo1 = _tc_pass3(z1, st2_e, eg2r, eb2r, n_total=AE, tile=640, rows=RH1)
    eo2 = _tc_pass3(z2, st2_e, eg2r, eb2r, n_total=AE, tile=640, rows=RH2)
    edge_out = jnp.concatenate([eo1, eo2], axis=0)

    vb_a = cW1[0:H, :]
    vb_b = cW1[H:2 * H, :]
    vc = cW1[2 * H:4 * H, :]
    vr = cW1[4 * H:, :]
    y5, st5 = _tc_cycle_pass1(p2_5, r2_5a, r2_5b, cycle_rep,
                              vc, vb_a, vb_b, vr, rows=A5, row_off=0)
    y6, st6 = _tc_cycle_pass1(p2_6, r2_6a, r2_6b, cycle_rep,
                              vc, vb_a, vb_b, vr,
                              rows=A6, row_off=A5 // 400)
    st_c = st5 + st6
    cg1r = cg1.reshape(1, -1)
    cb1r = cb1.reshape(1, -1)
    z5, st2_5 = _tc_pass2(y5, st_c, cW2, cg1r, cb1r,
                          n_total=AC, tile=400, rows=A5)
    z6, st2_6 = _tc_pass2(y6, st_c, cW2, cg1r, cb1r,
                          n_total=AC, tile=400, rows=A6)
    st2_c = st2_5 + st2_6
    cg2r = cg2.reshape(1, -1)
    cb2r = cb2.reshape(1, -1)
    co5 = _tc_pass3(z5, st2_c, cg2r, cb2r, n_total=AC, tile=400, rows=A5)
    co6 = _tc_pass3(z6, st2_c, cg2r, cb2r, n_total=AC, tile=400, rows=A6)
    cycle_out = jnp.concatenate([co5, co6], axis=0)

    return edge_out, cycle_out
